# baseline (speedup 1.0000x reference)
import jax
import jax.numpy as jnp
from jax import lax
from jax.experimental import pallas as pl
from jax.experimental.pallas import tpu as pltpu
from jax.experimental.pallas import tpu_sc as plsc

B = 16384
D = 64
NU = 100000
NI1 = 1001

NW = 32
BPW = B // NW
U_RANGE = 3200
I_RANGE = 32
LANES = 16


def _sc_body(user_hbm, pos_hbm, neg_hbm, label_hbm, utab_hbm,
             user_emb, batch_label, bm_user, bm_item,
             ubuf, bmu, bmi, sem_lab, sem_emb):
    nc = 2
    wid = lax.axis_index("s") * nc + lax.axis_index("c")
    base = wid * BPW

    pltpu.sync_copy(user_hbm, ubuf.at[pl.ds(0, B)])

    def fire_lab(r, _):
        idx = ubuf[pl.ds(base + r, LANES)][0]
        pltpu.make_async_copy(
            label_hbm.at[idx], batch_label.at[base + r], sem_lab).start()
        return 0

    lax.fori_loop(0, BPW, fire_lab, 0)

    def fire_emb(r, _):
        idx = ubuf[pl.ds(base + r, LANES)][0]
        pltpu.make_async_copy(
            utab_hbm.at[idx], user_emb.at[base + r], sem_emb).start()
        return 0

    lax.fori_loop(0, BPW, fire_emb, 0)

    lo_u = wid * U_RANGE
    zeros16 = jnp.zeros((LANES,), jnp.float32)
    ones16 = jnp.ones((LANES,), jnp.float32)

    def zero_bmu(i, _):
        bmu[pl.ds(i * LANES, LANES)] = zeros16
        return 0

    lax.fori_loop(0, U_RANGE // LANES, zero_bmu, 0)

    def mark_u(i, _):
        v = ubuf[pl.ds(i * LANES, LANES)]
        m = (v >= lo_u) & (v < lo_u + U_RANGE)
        plsc.store_scatter(bmu, [v - lo_u], ones16, mask=m)
        return 0

    lax.fori_loop(0, B // LANES, mark_u, 0)
    pltpu.sync_copy(bmu, bm_user.at[wid])

    lo_i = wid * I_RANGE
    bmi[pl.ds(0, LANES)] = zeros16
    bmi[pl.ds(LANES, LANES)] = zeros16

    def mark_i(i, _):
        v = ubuf[pl.ds(i * LANES, LANES)]
        m = (v >= lo_i) & (v < lo_i + I_RANGE)
        plsc.store_scatter(bmi, [v - lo_i], ones16, mask=m)
        return 0

    pltpu.sync_copy(pos_hbm, ubuf.at[pl.ds(0, B)])
    lax.fori_loop(0, B // LANES, mark_i, 0)
    pltpu.sync_copy(neg_hbm, ubuf.at[pl.ds(0, B)])
    lax.fori_loop(0, B // LANES, mark_i, 0)
    pltpu.sync_copy(bmi, bm_item.at[wid])

    def drain_lab(r, _):
        pltpu.make_async_copy(
            label_hbm.at[0], batch_label.at[0], sem_lab).wait()
        return 0

    lax.fori_loop(0, BPW, drain_lab, 0)

    def drain_emb(r, _):
        pltpu.make_async_copy(
            utab_hbm.at[0], user_emb.at[0], sem_emb).wait()
        return 0

    lax.fori_loop(0, BPW, drain_emb, 0)


def _sc_call(user, pos, neg, train_label, user_table):
    mesh = plsc.VectorSubcoreMesh(core_axis_name="c", subcore_axis_name="s")
    f = pl.kernel(
        _sc_body,
        out_type=[
            jax.ShapeDtypeStruct((B, D), jnp.float32),
            jax.ShapeDtypeStruct((B, NI1), jnp.float32),
            jax.ShapeDtypeStruct((NW, U_RANGE), jnp.float32),
            jax.ShapeDtypeStruct((NW, I_RANGE), jnp.float32),
        ],
        mesh=mesh,
        scratch_types=[
            pltpu.VMEM((B + LANES,), jnp.int32),
            pltpu.VMEM((U_RANGE,), jnp.float32),
            pltpu.VMEM((I_RANGE,), jnp.float32),
            pltpu.SemaphoreType.DMA,
            pltpu.SemaphoreType.DMA,
        ],
        compiler_params=pltpu.CompilerParams(needs_layout_passes=False),
    )
    return f(user, pos, neg, train_label, user_table)


BB = 256
GRID = B // BB
UB = 1568


def _tc_body(lab_ref, itab_ref, utab_ref, bmu_ref, bmi_ref, pos_ref, neg_ref,
             com_ref, aggu_ref, aggi_ref, pemb_ref, nemb_ref):
    lab = lab_ref[...]
    itab = itab_ref[...]
    com = jnp.dot(lab, itab, preferred_element_type=jnp.float32)
    num = jnp.sum(lab, axis=1, keepdims=True)
    com_ref[...] = com / num

    iota_k = lax.broadcasted_iota(jnp.int32, (BB, NI1), 1)
    pidx = pos_ref[0, 0, :]
    oh_p = (pidx[:, None] == iota_k).astype(jnp.float32)
    pemb_ref[...] = jnp.dot(oh_p, itab, preferred_element_type=jnp.float32)
    nidx = neg_ref[0, 0, :]
    oh_n = (nidx[:, None] == iota_k).astype(jnp.float32)
    nemb_ref[...] = jnp.dot(oh_n, itab, preferred_element_type=jnp.float32)

    aggu_ref[...] = utab_ref[...] * bmu_ref[...]

    @pl.when(pl.program_id(0) == 0)
    def _():
        aggi_ref[...] = itab * bmi_ref[...]


def _tc_call(batch_label, item_table, user_table, bmu, bmi, pos3, neg3):
    return pl.pallas_call(
        _tc_body,
        grid=(GRID,),
        in_specs=[
            pl.BlockSpec((BB, NI1), lambda i: (i, 0)),
            pl.BlockSpec((NI1, D), lambda i: (0, 0)),
            pl.BlockSpec((UB, D), lambda i: (i, 0)),
            pl.BlockSpec((UB, 1), lambda i: (i, 0)),
            pl.BlockSpec((NI1, 1), lambda i: (0, 0)),
            pl.BlockSpec((1, 1, BB), lambda i: (i, 0, 0)),
            pl.BlockSpec((1, 1, BB), lambda i: (i, 0, 0)),
        ],
        out_specs=[
            pl.BlockSpec((BB, D), lambda i: (i, 0)),
            pl.BlockSpec((UB, D), lambda i: (i, 0)),
            pl.BlockSpec((NI1, D), lambda i: (0, 0)),
            pl.BlockSpec((BB, D), lambda i: (i, 0)),
            pl.BlockSpec((BB, D), lambda i: (i, 0)),
        ],
        out_shape=[
            jax.ShapeDtypeStruct((B, D), jnp.float32),
            jax.ShapeDtypeStruct((NU, D), jnp.float32),
            jax.ShapeDtypeStruct((NI1, D), jnp.float32),
            jax.ShapeDtypeStruct((B, D), jnp.float32),
            jax.ShapeDtypeStruct((B, D), jnp.float32),
        ],
    )(batch_label, item_table, user_table, bmu, bmi, pos3, neg3)


@jax.jit
def kernel(user, pos, neg, train_label, user_table, item_table, agg_user, agg_item):
    del agg_user, agg_item
    user_emb, batch_label, bm_u, bm_i = _sc_call(
        user, pos, neg, train_label, user_table)
    bmu = bm_u.reshape(NW * U_RANGE, 1)
    bmi = bm_i.reshape(NW * I_RANGE, 1)[:NI1]
    pos3 = pos.reshape(GRID, 1, BB)
    neg3 = neg.reshape(GRID, 1, BB)
    pos_i_com, agg_user_o, agg_item_o, pos_emb, neg_emb = _tc_call(
        batch_label, item_table, user_table, bmu, bmi, pos3, neg3)
    return (user_emb, pos_emb, neg_emb, pos_i_com, agg_user_o, agg_item_o)


# speedup vs baseline: 3.6902x; 3.6902x over previous
import jax
import jax.numpy as jnp
from jax import lax
from jax.experimental import pallas as pl
from jax.experimental.pallas import tpu as pltpu
from jax.experimental.pallas import tpu_sc as plsc

B = 16384
D = 64
NU = 100000
NI1 = 1001
KA = 896
KB = NI1 - KA
KBP = 128

NW = 32
BPW = B // NW
U_RANGE = 3200
I_RANGE = 32
LANES = 16

LCH = 32
NLCH = BPW // LCH
WCH = 128
NWCH = BPW // WCH


def _pipelined_gather(src_ref, dst_hbm, idx_ref, nchunks, chunk, base,
                      bufs, gsems, wsems):
    def g(c):
        return pltpu.make_async_copy(
            src_ref.at[idx_ref.at[c]], bufs[c % 2], gsems[c % 2])

    def w(c):
        return pltpu.make_async_copy(
            bufs[c % 2], dst_hbm.at[pl.ds(base + c * chunk, chunk)],
            wsems[c % 2])

    for c in range(nchunks):
        if c >= 2:
            w(c - 2).wait()
        g(c).start()
        if c >= 1:
            g(c - 1).wait()
            w(c - 1).start()
    g(nchunks - 1).wait()
    w(nchunks - 1).start()
    w(nchunks - 2).wait()
    w(nchunks - 1).wait()


def _sc_body(user_hbm, pos_hbm, neg_hbm, label_hbm, tail_hbm, utabp_hbm,
             uembp, labA, labB, bm_user, bm_item,
             ubuf, bmu, bmi, idxl, idxw, lbuf0, lbuf1, wbuf0, wbuf1,
             gsem0, gsem1, wsem0, wsem1):
    nc = 2
    wid = lax.axis_index("s") * nc + lax.axis_index("c")
    base = wid * BPW

    for c in range(NLCH):
        pltpu.sync_copy(user_hbm.at[pl.ds(base + c * LCH, LCH)], idxl.at[c])
    for j in range(NWCH):
        pltpu.sync_copy(user_hbm.at[pl.ds(base + j * WCH, WCH)], idxw.at[j])

    _pipelined_gather(label_hbm.at[:, pl.ds(0, KA)], labA, idxl,
                      NLCH, LCH, base, (lbuf0, lbuf1),
                      (gsem0, gsem1), (wsem0, wsem1))
    _pipelined_gather(tail_hbm, labB, idxw, NWCH, WCH, base,
                      (wbuf0, wbuf1), (gsem0, gsem1), (wsem0, wsem1))
    _pipelined_gather(utabp_hbm, uembp, idxw, NWCH, WCH, base,
                      (wbuf0, wbuf1), (gsem0, gsem1), (wsem0, wsem1))

    lo_u = wid * U_RANGE
    zeros16 = jnp.zeros((LANES,), jnp.float32)
    ones16 = jnp.ones((LANES,), jnp.float32)

    pltpu.sync_copy(user_hbm, ubuf)

    def zero_bmu(i, _):
        bmu[pl.ds(i * LANES, LANES)] = zeros16
        return 0

    lax.fori_loop(0, U_RANGE // LANES, zero_bmu, 0)

    def mark_u(i, _):
        v = ubuf[pl.ds(i * LANES, LANES)]
        m = (v >= lo_u) & (v < lo_u + U_RANGE)
        plsc.store_scatter(bmu, [v - lo_u], ones16, mask=m)
        return 0

    lax.fori_loop(0, B // LANES, mark_u, 0)
    pltpu.sync_copy(bmu, bm_user.at[wid])

    lo_i = wid * I_RANGE
    bmi[pl.ds(0, LANES)] = zeros16
    bmi[pl.ds(LANES, LANES)] = zeros16

    def mark_i(i, _):
        v = ubuf[pl.ds(i * LANES, LANES)]
        m = (v >= lo_i) & (v < lo_i + I_RANGE)
        plsc.store_scatter(bmi, [v - lo_i], ones16, mask=m)
        return 0

    pltpu.sync_copy(pos_hbm, ubuf)
    lax.fori_loop(0, B // LANES, mark_i, 0)
    pltpu.sync_copy(neg_hbm, ubuf)
    lax.fori_loop(0, B // LANES, mark_i, 0)
    pltpu.sync_copy(bmi, bm_item.at[wid])


def _sc_call(user, pos, neg, train_label, tail_pad, utab_pad):
    mesh = plsc.VectorSubcoreMesh(core_axis_name="c", subcore_axis_name="s")
    f = pl.kernel(
        _sc_body,
        out_type=[
            jax.ShapeDtypeStruct((B, KBP), jnp.float32),
            jax.ShapeDtypeStruct((B, KA), jnp.float32),
            jax.ShapeDtypeStruct((B, KBP), jnp.float32),
            jax.ShapeDtypeStruct((NW, U_RANGE), jnp.float32),
            jax.ShapeDtypeStruct((NW, I_RANGE), jnp.float32),
        ],
        mesh=mesh,
        scratch_types=[
            pltpu.VMEM((B,), jnp.int32),
            pltpu.VMEM((U_RANGE,), jnp.float32),
            pltpu.VMEM((I_RANGE,), jnp.float32),
            pltpu.VMEM((NLCH, LCH), jnp.int32),
            pltpu.VMEM((NWCH, WCH), jnp.int32),
            pltpu.VMEM((LCH, KA), jnp.float32),
            pltpu.VMEM((LCH, KA), jnp.float32),
            pltpu.VMEM((WCH, KBP), jnp.float32),
            pltpu.VMEM((WCH, KBP), jnp.float32),
            pltpu.SemaphoreType.DMA,
            pltpu.SemaphoreType.DMA,
            pltpu.SemaphoreType.DMA,
            pltpu.SemaphoreType.DMA,
        ],
        compiler_params=pltpu.CompilerParams(needs_layout_passes=False),
    )
    return f(user, pos, neg, train_label, tail_pad, utab_pad)


BB = 256
GRID = B // BB
UB = 1568


def _tc_body(labA_ref, labB_ref, itemA_ref, itemB_ref, itab_ref,
             utab_ref, bmu_ref, bmi_ref, pos_ref, neg_ref,
             com_ref, aggu_ref, aggi_ref, pemb_ref, nemb_ref):
    labA = labA_ref[...]
    labB = labB_ref[...]
    itab = itab_ref[...]
    com = (jnp.dot(labA, itemA_ref[...], preferred_element_type=jnp.float32)
           + jnp.dot(labB, itemB_ref[...], preferred_element_type=jnp.float32))
    num = (jnp.sum(labA, axis=1, keepdims=True)
           + jnp.sum(labB, axis=1, keepdims=True))
    com_ref[...] = com / num

    iota_k = lax.broadcasted_iota(jnp.int32, (BB, NI1), 1)
    pidx = pos_ref[0, 0, :]
    oh_p = (pidx[:, None] == iota_k).astype(jnp.float32)
    pemb_ref[...] = jnp.dot(oh_p, itab, preferred_element_type=jnp.float32)
    nidx = neg_ref[0, 0, :]
    oh_n = (nidx[:, None] == iota_k).astype(jnp.float32)
    nemb_ref[...] = jnp.dot(oh_n, itab, preferred_element_type=jnp.float32)

    aggu_ref[...] = utab_ref[...] * bmu_ref[...]

    @pl.when(pl.program_id(0) == 0)
    def _():
        aggi_ref[...] = itab * bmi_ref[...]


def _tc_call(labA, labB, itemA, itemB, item_table, user_table,
             bmu, bmi, pos3, neg3):
    return pl.pallas_call(
        _tc_body,
        grid=(GRID,),
        in_specs=[
            pl.BlockSpec((BB, KA), lambda i: (i, 0)),
            pl.BlockSpec((BB, KBP), lambda i: (i, 0)),
            pl.BlockSpec((KA, D), lambda i: (0, 0)),
            pl.BlockSpec((KBP, D), lambda i: (0, 0)),
            pl.BlockSpec((NI1, D), lambda i: (0, 0)),
            pl.BlockSpec((UB, D), lambda i: (i, 0)),
            pl.BlockSpec((UB, 1), lambda i: (i, 0)),
            pl.BlockSpec((NI1, 1), lambda i: (0, 0)),
            pl.BlockSpec((1, 1, BB), lambda i: (i, 0, 0)),
            pl.BlockSpec((1, 1, BB), lambda i: (i, 0, 0)),
        ],
        out_specs=[
            pl.BlockSpec((BB, D), lambda i: (i, 0)),
            pl.BlockSpec((UB, D), lambda i: (i, 0)),
            pl.BlockSpec((NI1, D), lambda i: (0, 0)),
            pl.BlockSpec((BB, D), lambda i: (i, 0)),
            pl.BlockSpec((BB, D), lambda i: (i, 0)),
        ],
        out_shape=[
            jax.ShapeDtypeStruct((B, D), jnp.float32),
            jax.ShapeDtypeStruct((NU, D), jnp.float32),
            jax.ShapeDtypeStruct((NI1, D), jnp.float32),
            jax.ShapeDtypeStruct((B, D), jnp.float32),
            jax.ShapeDtypeStruct((B, D), jnp.float32),
        ],
    )(labA, labB, itemA, itemB, item_table, user_table, bmu, bmi, pos3, neg3)


@jax.jit
def kernel(user, pos, neg, train_label, user_table, item_table, agg_user, agg_item):
    del agg_user, agg_item
    tail_pad = jnp.pad(lax.slice(train_label, (0, KA), (NU, NI1)),
                       ((0, 0), (0, KBP - KB)))
    utab_pad = jnp.pad(user_table, ((0, 0), (0, KBP - D)))
    uembp, labA, labB, bm_u, bm_i = _sc_call(
        user, pos, neg, train_label, tail_pad, utab_pad)
    user_emb = lax.slice(uembp, (0, 0), (B, D))
    itemA = lax.slice(item_table, (0, 0), (KA, D))
    itemB = jnp.pad(lax.slice(item_table, (KA, 0), (NI1, D)),
                    ((0, KBP - KB), (0, 0)))
    bmu = bm_u.reshape(NW * U_RANGE, 1)
    bmi = bm_i.reshape(NW * I_RANGE, 1)[:NI1]
    pos3 = pos.reshape(GRID, 1, BB)
    neg3 = neg.reshape(GRID, 1, BB)
    pos_i_com, agg_user_o, agg_item_o, pos_emb, neg_emb = _tc_call(
        labA, labB, itemA, itemB, item_table, user_table, bmu, bmi, pos3, neg3)
    return (user_emb, pos_emb, neg_emb, pos_i_com, agg_user_o, agg_item_o)


# speedup vs baseline: 3.8065x; 1.0315x over previous
import jax
import jax.numpy as jnp
from jax import lax
from jax.experimental import pallas as pl
from jax.experimental.pallas import tpu as pltpu
from jax.experimental.pallas import tpu_sc as plsc

B = 16384
D = 64
NU = 100000
NI1 = 1001
KA = 896
KB = NI1 - KA
KBP = 128

NW = 32
BPW = B // NW
U_RANGE = 3200
I_RANGE = 32
LANES = 16

LCH = 32
NLCH = BPW // LCH
WCH = 128
NWCH = BPW // WCH


def _pipelined_gather(src_ref, dst_hbm, idx_ref, nchunks, chunk, base,
                      bufs, gsems, wsems):
    def g(c):
        return pltpu.make_async_copy(
            src_ref.at[idx_ref.at[c]], bufs[c % 2], gsems[c % 2])

    def w(c):
        return pltpu.make_async_copy(
            bufs[c % 2], dst_hbm.at[pl.ds(base + c * chunk, chunk)],
            wsems[c % 2])

    for c in range(nchunks):
        if c >= 2:
            w(c - 2).wait()
        g(c).start()
        if c >= 1:
            g(c - 1).wait()
            w(c - 1).start()
    g(nchunks - 1).wait()
    w(nchunks - 1).start()
    w(nchunks - 2).wait()
    w(nchunks - 1).wait()


def _sc_body(user_hbm, pos_hbm, neg_hbm, label_hbm, tail_hbm, utabp_hbm,
             uembp, labA, labB, bm_user, bm_item,
             ubuf, bmu, bmi, idxl, idxw, lbuf0, lbuf1, wbuf0, wbuf1,
             gsem0, gsem1, wsem0, wsem1):
    nc = 2
    wid = lax.axis_index("s") * nc + lax.axis_index("c")
    base = wid * BPW

    for c in range(NLCH):
        pltpu.sync_copy(user_hbm.at[pl.ds(base + c * LCH, LCH)], idxl.at[c])
    for j in range(NWCH):
        pltpu.sync_copy(user_hbm.at[pl.ds(base + j * WCH, WCH)], idxw.at[j])

    _pipelined_gather(label_hbm.at[:, pl.ds(0, KA)], labA, idxl,
                      NLCH, LCH, base, (lbuf0, lbuf1),
                      (gsem0, gsem1), (wsem0, wsem1))
    _pipelined_gather(tail_hbm, labB, idxw, NWCH, WCH, base,
                      (wbuf0, wbuf1), (gsem0, gsem1), (wsem0, wsem1))
    _pipelined_gather(utabp_hbm, uembp, idxw, NWCH, WCH, base,
                      (wbuf0, wbuf1), (gsem0, gsem1), (wsem0, wsem1))

    lo_u = wid * U_RANGE
    zeros16 = jnp.zeros((LANES,), jnp.float32)
    ones16 = jnp.ones((LANES,), jnp.float32)

    pltpu.sync_copy(user_hbm, ubuf)

    def zero_bmu(i, _):
        bmu[pl.ds(i * LANES, LANES)] = zeros16
        return 0

    lax.fori_loop(0, U_RANGE // LANES, zero_bmu, 0)

    def mark_u(i, _):
        v = ubuf[pl.ds(i * LANES, LANES)]
        m = (v >= lo_u) & (v < lo_u + U_RANGE)
        plsc.store_scatter(bmu, [v - lo_u], ones16, mask=m)
        return 0

    lax.fori_loop(0, B // LANES, mark_u, 0)
    pltpu.sync_copy(bmu, bm_user.at[wid])

    lo_i = wid * I_RANGE
    bmi[pl.ds(0, LANES)] = zeros16
    bmi[pl.ds(LANES, LANES)] = zeros16

    def mark_i(i, _):
        v = ubuf[pl.ds(i * LANES, LANES)]
        m = (v >= lo_i) & (v < lo_i + I_RANGE)
        plsc.store_scatter(bmi, [v - lo_i], ones16, mask=m)
        return 0

    pltpu.sync_copy(pos_hbm, ubuf)
    lax.fori_loop(0, B // LANES, mark_i, 0)
    pltpu.sync_copy(neg_hbm, ubuf)
    lax.fori_loop(0, B // LANES, mark_i, 0)
    pltpu.sync_copy(bmi, bm_item.at[wid])


def _sc_call(user, pos, neg, train_label, tail_pad, utab_pad):
    mesh = plsc.VectorSubcoreMesh(core_axis_name="c", subcore_axis_name="s")
    f = pl.kernel(
        _sc_body,
        out_type=[
            jax.ShapeDtypeStruct((B, KBP), jnp.float32),
            jax.ShapeDtypeStruct((B, KA), jnp.float32),
            jax.ShapeDtypeStruct((B, KBP), jnp.float32),
            jax.ShapeDtypeStruct((NW, U_RANGE), jnp.float32),
            jax.ShapeDtypeStruct((NW, I_RANGE), jnp.float32),
        ],
        mesh=mesh,
        scratch_types=[
            pltpu.VMEM((B,), jnp.int32),
            pltpu.VMEM((U_RANGE,), jnp.float32),
            pltpu.VMEM((I_RANGE,), jnp.float32),
            pltpu.VMEM((NLCH, LCH), jnp.int32),
            pltpu.VMEM((NWCH, WCH), jnp.int32),
            pltpu.VMEM((LCH, KA), jnp.float32),
            pltpu.VMEM((LCH, KA), jnp.float32),
            pltpu.VMEM((WCH, KBP), jnp.float32),
            pltpu.VMEM((WCH, KBP), jnp.float32),
            pltpu.SemaphoreType.DMA,
            pltpu.SemaphoreType.DMA,
            pltpu.SemaphoreType.DMA,
            pltpu.SemaphoreType.DMA,
        ],
        compiler_params=pltpu.CompilerParams(needs_layout_passes=False),
    )
    return f(user, pos, neg, train_label, tail_pad, utab_pad)


PB = 2048
PGRID = 49


def _prep_body(tail_ref, utab_ref, tailp_ref, utabp_ref):
    lane = lax.broadcasted_iota(jnp.int32, (PB, KBP), 1)
    tailp_ref[...] = jnp.where(lane < KB, tail_ref[...], 0.0)
    utabp_ref[...] = jnp.concatenate(
        [utab_ref[...], jnp.zeros((PB, KBP - D), jnp.float32)], axis=1)


def _prep_call(train_label, user_table):
    return pl.pallas_call(
        _prep_body,
        grid=(PGRID,),
        in_specs=[
            pl.BlockSpec((PB, KBP), lambda i: (i, 7)),
            pl.BlockSpec((PB, D), lambda i: (i, 0)),
        ],
        out_specs=[
            pl.BlockSpec((PB, KBP), lambda i: (i, 0)),
            pl.BlockSpec((PB, KBP), lambda i: (i, 0)),
        ],
        out_shape=[
            jax.ShapeDtypeStruct((NU, KBP), jnp.float32),
            jax.ShapeDtypeStruct((NU, KBP), jnp.float32),
        ],
    )(train_label, user_table)


BB = 256
GRID = B // BB
UB = 1568


def _tc_body(labA_ref, labB_ref, itemA_ref, itemB_ref, itab_ref,
             utab_ref, bmu_ref, bmi_ref, pos_ref, neg_ref, uembp_ref,
             com_ref, aggu_ref, aggi_ref, pemb_ref, nemb_ref, uemb_ref):
    labA = labA_ref[...]
    labB = labB_ref[...]
    itab = itab_ref[...]
    com = (jnp.dot(labA, itemA_ref[...], preferred_element_type=jnp.float32)
           + jnp.dot(labB, itemB_ref[...], preferred_element_type=jnp.float32))
    num = (jnp.sum(labA, axis=1, keepdims=True)
           + jnp.sum(labB, axis=1, keepdims=True))
    com_ref[...] = com / num

    iota_k = lax.broadcasted_iota(jnp.int32, (BB, NI1), 1)
    pidx = pos_ref[0, 0, :]
    oh_p = (pidx[:, None] == iota_k).astype(jnp.float32)
    pemb_ref[...] = jnp.dot(oh_p, itab, preferred_element_type=jnp.float32)
    nidx = neg_ref[0, 0, :]
    oh_n = (nidx[:, None] == iota_k).astype(jnp.float32)
    nemb_ref[...] = jnp.dot(oh_n, itab, preferred_element_type=jnp.float32)

    aggu_ref[...] = utab_ref[...] * bmu_ref[...]
    uemb_ref[...] = uembp_ref[:, :D]

    @pl.when(pl.program_id(0) == 0)
    def _():
        aggi_ref[...] = itab * bmi_ref[...]


def _tc_call(labA, labB, itemA, itemB, item_table, user_table,
             bmu, bmi, pos3, neg3, uembp):
    return pl.pallas_call(
        _tc_body,
        grid=(GRID,),
        in_specs=[
            pl.BlockSpec((BB, KA), lambda i: (i, 0)),
            pl.BlockSpec((BB, KBP), lambda i: (i, 0)),
            pl.BlockSpec((KA, D), lambda i: (0, 0)),
            pl.BlockSpec((KBP, D), lambda i: (0, 0)),
            pl.BlockSpec((NI1, D), lambda i: (0, 0)),
            pl.BlockSpec((UB, D), lambda i: (i, 0)),
            pl.BlockSpec((UB, 1), lambda i: (i, 0)),
            pl.BlockSpec((NI1, 1), lambda i: (0, 0)),
            pl.BlockSpec((1, 1, BB), lambda i: (i, 0, 0)),
            pl.BlockSpec((1, 1, BB), lambda i: (i, 0, 0)),
            pl.BlockSpec((BB, KBP), lambda i: (i, 0)),
        ],
        out_specs=[
            pl.BlockSpec((BB, D), lambda i: (i, 0)),
            pl.BlockSpec((UB, D), lambda i: (i, 0)),
            pl.BlockSpec((NI1, D), lambda i: (0, 0)),
            pl.BlockSpec((BB, D), lambda i: (i, 0)),
            pl.BlockSpec((BB, D), lambda i: (i, 0)),
            pl.BlockSpec((BB, D), lambda i: (i, 0)),
        ],
        out_shape=[
            jax.ShapeDtypeStruct((B, D), jnp.float32),
            jax.ShapeDtypeStruct((NU, D), jnp.float32),
            jax.ShapeDtypeStruct((NI1, D), jnp.float32),
            jax.ShapeDtypeStruct((B, D), jnp.float32),
            jax.ShapeDtypeStruct((B, D), jnp.float32),
            jax.ShapeDtypeStruct((B, D), jnp.float32),
        ],
    )(labA, labB, itemA, itemB, item_table, user_table, bmu, bmi, pos3, neg3,
      uembp)


@jax.jit
def kernel(user, pos, neg, train_label, user_table, item_table, agg_user, agg_item):
    del agg_user, agg_item
    tail_pad, utab_pad = _prep_call(train_label, user_table)
    uembp, labA, labB, bm_u, bm_i = _sc_call(
        user, pos, neg, train_label, tail_pad, utab_pad)
    itemA = lax.slice(item_table, (0, 0), (KA, D))
    itemB = jnp.pad(lax.slice(item_table, (KA, 0), (NI1, D)),
                    ((0, KBP - KB), (0, 0)))
    bmu = bm_u.reshape(NW * U_RANGE, 1)
    bmi = bm_i.reshape(NW * I_RANGE, 1)[:NI1]
    pos3 = pos.reshape(GRID, 1, BB)
    neg3 = neg.reshape(GRID, 1, BB)
    pos_i_com, agg_user_o, agg_item_o, pos_emb, neg_emb, user_emb = _tc_call(
        labA, labB, itemA, itemB, item_table, user_table, bmu, bmi, pos3, neg3,
        uembp)
    return (user_emb, pos_emb, neg_emb, pos_i_com, agg_user_o, agg_item_o)


# speedup vs baseline: 6.5545x; 1.7219x over previous
import jax
import jax.numpy as jnp
from jax import lax
from jax.experimental import pallas as pl
from jax.experimental.pallas import tpu as pltpu
from jax.experimental.pallas import tpu_sc as plsc

B = 16384
D = 64
NU = 100000
NI1 = 1001
GW = 256

NW = 32
BPW = B // NW
U_RANGE = 3200
I_RANGE = 128
NIW = 8
LANES = 16

WCH = 128
NWCH = BPW // WCH

PB = 512
PGRID = 196
NUP = PGRID * PB


def _prep_body(labt_ref, itab_ref, utab_ref, pug_ref):
    labt = labt_ref[...]
    pblk = lax.dot_general(labt, itab_ref[...], (((0,), (0,)), ((), ())),
                           preferred_element_type=jnp.float32)
    num = jnp.sum(labt, axis=0)[:, None]
    pug_ref[...] = jnp.concatenate([
        pblk, num, jnp.zeros((PB, 63), jnp.float32),
        utab_ref[...], jnp.zeros((PB, 64), jnp.float32)], axis=1)


def _prep_call(labt, item_table, user_table):
    return pl.pallas_call(
        _prep_body,
        grid=(PGRID,),
        in_specs=[
            pl.BlockSpec((NI1, PB), lambda i: (0, i)),
            pl.BlockSpec((NI1, D), lambda i: (0, 0)),
            pl.BlockSpec((PB, D), lambda i: (i, 0)),
        ],
        out_specs=[pl.BlockSpec((PB, GW), lambda i: (i, 0))],
        out_shape=[jax.ShapeDtypeStruct((NUP, GW), jnp.float32)],
    )(labt, item_table, user_table)


def _pipelined_gather(src_ref, dst_hbm, idx_ref, nchunks, chunk, base,
                      bufs, gsems, wsems):
    def g(c):
        return pltpu.make_async_copy(
            src_ref.at[idx_ref.at[c]], bufs[c % 2], gsems[c % 2])

    def w(c):
        return pltpu.make_async_copy(
            bufs[c % 2], dst_hbm.at[pl.ds(base + c * chunk, chunk)],
            wsems[c % 2])

    for c in range(nchunks):
        if c >= 2:
            w(c - 2).wait()
        g(c).start()
        if c >= 1:
            g(c - 1).wait()
            w(c - 1).start()
    g(nchunks - 1).wait()
    w(nchunks - 1).start()
    w(nchunks - 2).wait()
    w(nchunks - 1).wait()


def _sc_body(user_hbm, pos_hbm, neg_hbm, pug_hbm,
             pugg, bm_user, bm_item,
             ubuf, bmu, bmi, idxw, wbuf0, wbuf1,
             gsem0, gsem1, wsem0, wsem1):
    nc = 2
    wid = lax.axis_index("s") * nc + lax.axis_index("c")
    base = wid * BPW

    for j in range(NWCH):
        pltpu.sync_copy(user_hbm.at[pl.ds(base + j * WCH, WCH)], idxw.at[j])

    _pipelined_gather(pug_hbm, pugg, idxw, NWCH, WCH, base,
                      (wbuf0, wbuf1), (gsem0, gsem1), (wsem0, wsem1))

    lo_u = wid * U_RANGE
    zeros16 = jnp.zeros((LANES,), jnp.float32)
    ones16 = jnp.ones((LANES,), jnp.float32)

    pltpu.sync_copy(user_hbm, ubuf)

    def zero_bmu(i, _):
        bmu[pl.ds(i * LANES, LANES)] = zeros16
        return 0

    lax.fori_loop(0, U_RANGE // LANES, zero_bmu, 0)

    def mark_u(i, _):
        v = ubuf[pl.ds(i * LANES, LANES)]
        m = (v >= lo_u) & (v < lo_u + U_RANGE)
        plsc.store_scatter(bmu, [v - lo_u], ones16, mask=m)
        return 0

    lax.fori_loop(0, B // LANES, mark_u, 0)
    pltpu.sync_copy(bmu, bm_user.at[0, pl.ds(wid * U_RANGE, U_RANGE)])

    lo_i = wid * I_RANGE

    def zero_bmi(i, _):
        bmi[pl.ds(i * LANES, LANES)] = zeros16
        return 0

    def mark_i(i, _):
        v = ubuf[pl.ds(i * LANES, LANES)]
        m = (v >= lo_i) & (v < lo_i + I_RANGE)
        plsc.store_scatter(bmi, [v - lo_i], ones16, mask=m)
        return 0

    lax.fori_loop(0, I_RANGE // LANES, zero_bmi, 0)
    pltpu.sync_copy(pos_hbm, ubuf)

    @pl.when(wid < NIW)
    def _():
        lax.fori_loop(0, B // LANES, mark_i, 0)

    pltpu.sync_copy(neg_hbm, ubuf)

    @pl.when(wid < NIW)
    def _():
        lax.fori_loop(0, B // LANES, mark_i, 0)
        pltpu.sync_copy(bmi, bm_item.at[0, pl.ds(wid * I_RANGE, I_RANGE)])


def _sc_call(user, pos, neg, pug):
    mesh = plsc.VectorSubcoreMesh(core_axis_name="c", subcore_axis_name="s")
    f = pl.kernel(
        _sc_body,
        out_type=[
            jax.ShapeDtypeStruct((B, GW), jnp.float32),
            jax.ShapeDtypeStruct((1, NW * U_RANGE), jnp.float32),
            jax.ShapeDtypeStruct((1, NIW * I_RANGE), jnp.float32),
        ],
        mesh=mesh,
        scratch_types=[
            pltpu.VMEM((B,), jnp.int32),
            pltpu.VMEM((U_RANGE,), jnp.float32),
            pltpu.VMEM((I_RANGE,), jnp.float32),
            pltpu.VMEM((NWCH, WCH), jnp.int32),
            pltpu.VMEM((WCH, GW), jnp.float32),
            pltpu.VMEM((WCH, GW), jnp.float32),
            pltpu.SemaphoreType.DMA,
            pltpu.SemaphoreType.DMA,
            pltpu.SemaphoreType.DMA,
            pltpu.SemaphoreType.DMA,
        ],
        compiler_params=pltpu.CompilerParams(needs_layout_passes=False),
    )
    return f(user, pos, neg, pug)


BB = 256
GRID = B // BB
AB = 2048
AGRID = 49


def _tc_body(itabt_ref, pos_ref, neg_ref, pugg_ref,
             comt_ref, pembt_ref, nembt_ref, uembt_ref):
    itabt = itabt_ref[...]
    t = jnp.transpose(pugg_ref[...])
    comt_ref[...] = t[:D, :] / t[D:D + 1, :]
    uembt_ref[...] = t[GW // 2:GW // 2 + D, :]

    iota_k = lax.broadcasted_iota(jnp.int32, (NI1, BB), 0)
    pidx = pos_ref[0, 0, :]
    oh_p = (pidx[None, :] == iota_k).astype(jnp.float32)
    pembt_ref[...] = jnp.dot(itabt, oh_p, preferred_element_type=jnp.float32)
    nidx = neg_ref[0, 0, :]
    oh_n = (nidx[None, :] == iota_k).astype(jnp.float32)
    nembt_ref[...] = jnp.dot(itabt, oh_n, preferred_element_type=jnp.float32)


def _tc_call(itabt, pos3, neg3, pugg):
    return pl.pallas_call(
        _tc_body,
        grid=(GRID,),
        in_specs=[
            pl.BlockSpec((D, NI1), lambda i: (0, 0)),
            pl.BlockSpec((1, 1, BB), lambda i: (i, 0, 0)),
            pl.BlockSpec((1, 1, BB), lambda i: (i, 0, 0)),
            pl.BlockSpec((BB, GW), lambda i: (i, 0)),
        ],
        out_specs=[
            pl.BlockSpec((D, BB), lambda i: (0, i)),
            pl.BlockSpec((D, BB), lambda i: (0, i)),
            pl.BlockSpec((D, BB), lambda i: (0, i)),
            pl.BlockSpec((D, BB), lambda i: (0, i)),
        ],
        out_shape=[
            jax.ShapeDtypeStruct((D, B), jnp.float32),
            jax.ShapeDtypeStruct((D, B), jnp.float32),
            jax.ShapeDtypeStruct((D, B), jnp.float32),
            jax.ShapeDtypeStruct((D, B), jnp.float32),
        ],
    )(itabt, pos3, neg3, pugg)


def _agg_body(utt_ref, bmu_ref, itabt_ref, bmi_ref, aggut_ref, aggit_ref):
    aggut_ref[...] = utt_ref[...] * bmu_ref[...]

    @pl.when(pl.program_id(0) == 0)
    def _():
        aggit_ref[...] = itabt_ref[...] * bmi_ref[:, :NI1]


def _agg_call(utt, bmu, itabt, bmi):
    return pl.pallas_call(
        _agg_body,
        grid=(AGRID,),
        in_specs=[
            pl.BlockSpec((D, AB), lambda i: (0, i)),
            pl.BlockSpec((1, AB), lambda i: (0, i)),
            pl.BlockSpec((D, NI1), lambda i: (0, 0)),
            pl.BlockSpec((1, NIW * I_RANGE), lambda i: (0, 0)),
        ],
        out_specs=[
            pl.BlockSpec((D, AB), lambda i: (0, i)),
            pl.BlockSpec((D, NI1), lambda i: (0, 0)),
        ],
        out_shape=[
            jax.ShapeDtypeStruct((D, NU), jnp.float32),
            jax.ShapeDtypeStruct((D, NI1), jnp.float32),
        ],
    )(utt, bmu, itabt, bmi)


@jax.jit
def kernel(user, pos, neg, train_label, user_table, item_table, agg_user, agg_item):
    del agg_user, agg_item
    labt = jnp.transpose(train_label)
    itabt = jnp.transpose(item_table)
    utt = jnp.transpose(user_table)
    (pug,) = _prep_call(labt, item_table, user_table)
    pugg, bm_u, bm_i = _sc_call(user, pos, neg, pug)
    pos3 = pos.reshape(GRID, 1, BB)
    neg3 = neg.reshape(GRID, 1, BB)
    comt, pembt, nembt, uembt = _tc_call(itabt, pos3, neg3, pugg)
    aggut, aggit = _agg_call(utt, bm_u, itabt, bm_i)
    return (jnp.transpose(uembt), jnp.transpose(pembt), jnp.transpose(nembt),
            jnp.transpose(comt), jnp.transpose(aggut), jnp.transpose(aggit))


# speedup vs baseline: 8.1696x; 1.2464x over previous
import jax
import jax.numpy as jnp
from jax import lax
from jax.experimental import pallas as pl
from jax.experimental.pallas import tpu as pltpu
from jax.experimental.pallas import tpu_sc as plsc

B = 16384
D = 64
NU = 100000
NI1 = 1001
GW = 128

NW = 32
BPW = B // NW
U_RANGE = 3200
I_RANGE = 128
NIW = 8
LANES = 16

WCH = 128
NWCH = BPW // WCH

PB = 1024
PGRID = 98
NUP = PGRID * PB


def _prep_body(labt_ref, itab_ref, utab_ref, pug_ref):
    labt = labt_ref[...]
    pblk = lax.dot_general(labt, itab_ref[...], (((0,), (0,)), ((), ())),
                           preferred_element_type=jnp.float32)
    num = jnp.sum(labt, axis=0)[:, None]
    pug_ref[...] = jnp.concatenate([pblk / num, utab_ref[...]], axis=1)


def _prep_call(labt, item_table, user_table):
    return pl.pallas_call(
        _prep_body,
        grid=(PGRID,),
        in_specs=[
            pl.BlockSpec((NI1, PB), lambda i: (0, i)),
            pl.BlockSpec((NI1, D), lambda i: (0, 0)),
            pl.BlockSpec((PB, D), lambda i: (i, 0)),
        ],
        out_specs=[pl.BlockSpec((PB, GW), lambda i: (i, 0))],
        out_shape=[jax.ShapeDtypeStruct((NUP, GW), jnp.float32)],
    )(labt, item_table, user_table)


def _pipelined_gather(src_ref, dst_hbm, idx_ref, nchunks, chunk, base,
                      bufs, gsems, wsems):
    def g(c):
        return pltpu.make_async_copy(
            src_ref.at[idx_ref.at[c]], bufs[c % 2], gsems[c % 2])

    def w(c):
        return pltpu.make_async_copy(
            bufs[c % 2], dst_hbm.at[pl.ds(base + c * chunk, chunk)],
            wsems[c % 2])

    for c in range(nchunks):
        if c >= 2:
            w(c - 2).wait()
        g(c).start()
        if c >= 1:
            g(c - 1).wait()
            w(c - 1).start()
    g(nchunks - 1).wait()
    w(nchunks - 1).start()
    w(nchunks - 2).wait()
    w(nchunks - 1).wait()


def _sc_body(user_hbm, pos_hbm, neg_hbm, pug_hbm,
             pugg, bm_user, bm_item,
             ubuf, bmu, bmi, idxw, wbuf0, wbuf1,
             gsem0, gsem1, wsem0, wsem1):
    nc = 2
    wid = lax.axis_index("s") * nc + lax.axis_index("c")
    base = wid * BPW

    for j in range(NWCH):
        pltpu.sync_copy(user_hbm.at[pl.ds(base + j * WCH, WCH)], idxw.at[j])

    _pipelined_gather(pug_hbm, pugg, idxw, NWCH, WCH, base,
                      (wbuf0, wbuf1), (gsem0, gsem1), (wsem0, wsem1))

    lo_u = wid * U_RANGE
    zeros16 = jnp.zeros((LANES,), jnp.float32)
    ones16 = jnp.ones((LANES,), jnp.float32)

    pltpu.sync_copy(user_hbm, ubuf)

    def zero_bmu(i, _):
        bmu[pl.ds(i * LANES, LANES)] = zeros16
        return 0

    lax.fori_loop(0, U_RANGE // LANES, zero_bmu, 0)

    def mark_u(i, _):
        v = ubuf[pl.ds(i * LANES, LANES)]
        m = (v >= lo_u) & (v < lo_u + U_RANGE)
        plsc.store_scatter(bmu, [v - lo_u], ones16, mask=m)
        return 0

    lax.fori_loop(0, B // LANES, mark_u, 0)
    pltpu.sync_copy(bmu, bm_user.at[0, pl.ds(wid * U_RANGE, U_RANGE)])

    lo_i = wid * I_RANGE

    def zero_bmi(i, _):
        bmi[pl.ds(i * LANES, LANES)] = zeros16
        return 0

    def mark_i(i, _):
        v = ubuf[pl.ds(i * LANES, LANES)]
        m = (v >= lo_i) & (v < lo_i + I_RANGE)
        plsc.store_scatter(bmi, [v - lo_i], ones16, mask=m)
        return 0

    lax.fori_loop(0, I_RANGE // LANES, zero_bmi, 0)
    pltpu.sync_copy(pos_hbm, ubuf)

    @pl.when(wid < NIW)
    def _():
        lax.fori_loop(0, B // LANES, mark_i, 0)

    pltpu.sync_copy(neg_hbm, ubuf)

    @pl.when(wid < NIW)
    def _():
        lax.fori_loop(0, B // LANES, mark_i, 0)
        pltpu.sync_copy(bmi, bm_item.at[0, pl.ds(wid * I_RANGE, I_RANGE)])


def _sc_call(user, pos, neg, pug):
    mesh = plsc.VectorSubcoreMesh(core_axis_name="c", subcore_axis_name="s")
    f = pl.kernel(
        _sc_body,
        out_type=[
            jax.ShapeDtypeStruct((B, GW), jnp.float32),
            jax.ShapeDtypeStruct((1, NW * U_RANGE), jnp.float32),
            jax.ShapeDtypeStruct((1, NIW * I_RANGE), jnp.float32),
        ],
        mesh=mesh,
        scratch_types=[
            pltpu.VMEM((B,), jnp.int32),
            pltpu.VMEM((U_RANGE,), jnp.float32),
            pltpu.VMEM((I_RANGE,), jnp.float32),
            pltpu.VMEM((NWCH, WCH), jnp.int32),
            pltpu.VMEM((WCH, GW), jnp.float32),
            pltpu.VMEM((WCH, GW), jnp.float32),
            pltpu.SemaphoreType.DMA,
            pltpu.SemaphoreType.DMA,
            pltpu.SemaphoreType.DMA,
            pltpu.SemaphoreType.DMA,
        ],
        compiler_params=pltpu.CompilerParams(needs_layout_passes=False),
    )
    return f(user, pos, neg, pug)


BB = 256
GRID = B // BB
AB = 4096
AGRID = 25


def _tc_body(itabt_ref, pos_ref, neg_ref, pugg_ref,
             comt_ref, pembt_ref, nembt_ref, uembt_ref):
    itabt = itabt_ref[...]
    t = jnp.transpose(pugg_ref[...])
    comt_ref[...] = t[:D, :]
    uembt_ref[...] = t[D:GW, :]

    iota_k = lax.broadcasted_iota(jnp.int32, (NI1, BB), 0)
    pidx = pos_ref[0, 0, :]
    oh_p = (pidx[None, :] == iota_k).astype(jnp.float32)
    pembt_ref[...] = jnp.dot(itabt, oh_p, preferred_element_type=jnp.float32)
    nidx = neg_ref[0, 0, :]
    oh_n = (nidx[None, :] == iota_k).astype(jnp.float32)
    nembt_ref[...] = jnp.dot(itabt, oh_n, preferred_element_type=jnp.float32)


def _tc_call(itabt, pos3, neg3, pugg):
    return pl.pallas_call(
        _tc_body,
        grid=(GRID,),
        in_specs=[
            pl.BlockSpec((D, NI1), lambda i: (0, 0)),
            pl.BlockSpec((1, 1, BB), lambda i: (i, 0, 0)),
            pl.BlockSpec((1, 1, BB), lambda i: (i, 0, 0)),
            pl.BlockSpec((BB, GW), lambda i: (i, 0)),
        ],
        out_specs=[
            pl.BlockSpec((D, BB), lambda i: (0, i)),
            pl.BlockSpec((D, BB), lambda i: (0, i)),
            pl.BlockSpec((D, BB), lambda i: (0, i)),
            pl.BlockSpec((D, BB), lambda i: (0, i)),
        ],
        out_shape=[
            jax.ShapeDtypeStruct((D, B), jnp.float32),
            jax.ShapeDtypeStruct((D, B), jnp.float32),
            jax.ShapeDtypeStruct((D, B), jnp.float32),
            jax.ShapeDtypeStruct((D, B), jnp.float32),
        ],
    )(itabt, pos3, neg3, pugg)


def _agg_body(utt_ref, bmu_ref, itabt_ref, bmi_ref, aggut_ref, aggit_ref):
    aggut_ref[...] = utt_ref[...] * bmu_ref[...]

    @pl.when(pl.program_id(0) == 0)
    def _():
        aggit_ref[...] = itabt_ref[...] * bmi_ref[:, :NI1]


def _agg_call(utt, bmu, itabt, bmi):
    return pl.pallas_call(
        _agg_body,
        grid=(AGRID,),
        in_specs=[
            pl.BlockSpec((D, AB), lambda i: (0, i)),
            pl.BlockSpec((1, AB), lambda i: (0, i)),
            pl.BlockSpec((D, NI1), lambda i: (0, 0)),
            pl.BlockSpec((1, NIW * I_RANGE), lambda i: (0, 0)),
        ],
        out_specs=[
            pl.BlockSpec((D, AB), lambda i: (0, i)),
            pl.BlockSpec((D, NI1), lambda i: (0, 0)),
        ],
        out_shape=[
            jax.ShapeDtypeStruct((D, NU), jnp.float32),
            jax.ShapeDtypeStruct((D, NI1), jnp.float32),
        ],
    )(utt, bmu, itabt, bmi)


@jax.jit
def kernel(user, pos, neg, train_label, user_table, item_table, agg_user, agg_item):
    del agg_user, agg_item
    labt = jnp.transpose(train_label)
    itabt = jnp.transpose(item_table)
    utt = jnp.transpose(user_table)
    (pug,) = _prep_call(labt, item_table, user_table)
    pugg, bm_u, bm_i = _sc_call(user, pos, neg, pug)
    pos3 = pos.reshape(GRID, 1, BB)
    neg3 = neg.reshape(GRID, 1, BB)
    comt, pembt, nembt, uembt = _tc_call(itabt, pos3, neg3, pugg)
    aggut, aggit = _agg_call(utt, bm_u, itabt, bm_i)
    return (jnp.transpose(uembt), jnp.transpose(pembt), jnp.transpose(nembt),
            jnp.transpose(comt), jnp.transpose(aggut), jnp.transpose(aggit))


# speedup vs baseline: 10.0583x; 1.2312x over previous
import jax
import jax.numpy as jnp
from jax import lax
from jax.experimental import pallas as pl
from jax.experimental.pallas import tpu as pltpu
from jax.experimental.pallas import tpu_sc as plsc

B = 16384
D = 64
NU = 100000
NI1 = 1001
GW = 128

NW = 32
BPW = B // NW
U_RANGE = 3200
I_RANGE = 128
NIW = 8
LANES = 16

WCH = 128
NWCH = BPW // WCH

PB = 2048
PGRID = 49
NUP = PGRID * PB


def _prep_body(labt_ref, itab_ref, utt_ref, pug_ref):
    labt = labt_ref[...]
    pblk = lax.dot_general(labt, itab_ref[...], (((0,), (0,)), ((), ())),
                           preferred_element_type=jnp.float32)
    num = jnp.sum(labt, axis=0)[:, None]
    urow = jnp.transpose(utt_ref[...])
    pug_ref[...] = jnp.concatenate([pblk / num, urow], axis=1)


def _prep_call(labt, item_table, utt):
    return pl.pallas_call(
        _prep_body,
        grid=(PGRID,),
        in_specs=[
            pl.BlockSpec((NI1, PB), lambda i: (0, i)),
            pl.BlockSpec((NI1, D), lambda i: (0, 0)),
            pl.BlockSpec((D, PB), lambda i: (0, i)),
        ],
        out_specs=[pl.BlockSpec((PB, GW), lambda i: (i, 0))],
        out_shape=[jax.ShapeDtypeStruct((NUP, GW), jnp.float32)],
    )(labt, item_table, utt)


def _pipelined_gather(src_ref, dst_hbm, idx_ref, nchunks, chunk, base,
                      bufs, gsems, wsems):
    def g(c):
        return pltpu.make_async_copy(
            src_ref.at[idx_ref.at[c]], bufs[c % 2], gsems[c % 2])

    def w(c):
        return pltpu.make_async_copy(
            bufs[c % 2], dst_hbm.at[pl.ds(base + c * chunk, chunk)],
            wsems[c % 2])

    for c in range(nchunks):
        if c >= 2:
            w(c - 2).wait()
        g(c).start()
        if c >= 1:
            g(c - 1).wait()
            w(c - 1).start()
    g(nchunks - 1).wait()
    w(nchunks - 1).start()
    w(nchunks - 2).wait()
    w(nchunks - 1).wait()


def _sc_body(user_hbm, pos_hbm, neg_hbm, pug_hbm,
             pugg, bm_user, bm_item,
             ubuf, bmu, bmi, idxw, wbuf0, wbuf1,
             gsem0, gsem1, wsem0, wsem1):
    nc = 2
    wid = lax.axis_index("s") * nc + lax.axis_index("c")
    base = wid * BPW

    for j in range(NWCH):
        pltpu.sync_copy(user_hbm.at[pl.ds(base + j * WCH, WCH)], idxw.at[j])

    _pipelined_gather(pug_hbm, pugg, idxw, NWCH, WCH, base,
                      (wbuf0, wbuf1), (gsem0, gsem1), (wsem0, wsem1))

    lo_u = wid * U_RANGE
    zeros16 = jnp.zeros((LANES,), jnp.float32)
    ones16 = jnp.ones((LANES,), jnp.float32)

    pltpu.sync_copy(user_hbm, ubuf)

    def zero_bmu(i, _):
        bmu[pl.ds(i * LANES, LANES)] = zeros16
        return 0

    lax.fori_loop(0, U_RANGE // LANES, zero_bmu, 0)

    def mark_u(i, _):
        v = ubuf[pl.ds(i * LANES, LANES)]
        m = (v >= lo_u) & (v < lo_u + U_RANGE)
        plsc.store_scatter(bmu, [v - lo_u], ones16, mask=m)
        return 0

    lax.fori_loop(0, B // LANES, mark_u, 0)
    pltpu.sync_copy(bmu, bm_user.at[0, pl.ds(wid * U_RANGE, U_RANGE)])

    lo_i = wid * I_RANGE

    def zero_bmi(i, _):
        bmi[pl.ds(i * LANES, LANES)] = zeros16
        return 0

    def mark_i(i, _):
        v = ubuf[pl.ds(i * LANES, LANES)]
        m = (v >= lo_i) & (v < lo_i + I_RANGE)
        plsc.store_scatter(bmi, [v - lo_i], ones16, mask=m)
        return 0

    lax.fori_loop(0, I_RANGE // LANES, zero_bmi, 0)
    pltpu.sync_copy(pos_hbm, ubuf)

    @pl.when(wid < NIW)
    def _():
        lax.fori_loop(0, B // LANES, mark_i, 0)

    pltpu.sync_copy(neg_hbm, ubuf)

    @pl.when(wid < NIW)
    def _():
        lax.fori_loop(0, B // LANES, mark_i, 0)
        pltpu.sync_copy(bmi, bm_item.at[0, pl.ds(wid * I_RANGE, I_RANGE)])


def _sc_call(user, pos, neg, pug):
    mesh = plsc.VectorSubcoreMesh(core_axis_name="c", subcore_axis_name="s")
    f = pl.kernel(
        _sc_body,
        out_type=[
            jax.ShapeDtypeStruct((B, GW), jnp.float32),
            jax.ShapeDtypeStruct((1, NW * U_RANGE), jnp.float32),
            jax.ShapeDtypeStruct((1, NIW * I_RANGE), jnp.float32),
        ],
        mesh=mesh,
        scratch_types=[
            pltpu.VMEM((B,), jnp.int32),
            pltpu.VMEM((U_RANGE,), jnp.float32),
            pltpu.VMEM((I_RANGE,), jnp.float32),
            pltpu.VMEM((NWCH, WCH), jnp.int32),
            pltpu.VMEM((WCH, GW), jnp.float32),
            pltpu.VMEM((WCH, GW), jnp.float32),
            pltpu.SemaphoreType.DMA,
            pltpu.SemaphoreType.DMA,
            pltpu.SemaphoreType.DMA,
            pltpu.SemaphoreType.DMA,
        ],
        compiler_params=pltpu.CompilerParams(needs_layout_passes=False),
    )
    return f(user, pos, neg, pug)


BB = 256
GRID = B // BB
AB = 4096
AGRID = 25


def _tc_body(itabt_ref, pos_ref, neg_ref, pugg_ref,
             comt_ref, pembt_ref, nembt_ref, uembt_ref):
    itabt = itabt_ref[...]
    t = jnp.transpose(pugg_ref[...])
    comt_ref[...] = t[:D, :]
    uembt_ref[...] = t[D:GW, :]

    iota_k = lax.broadcasted_iota(jnp.int32, (NI1, BB), 0)
    pidx = pos_ref[0, 0, :]
    oh_p = (pidx[None, :] == iota_k).astype(jnp.float32)
    pembt_ref[...] = jnp.dot(itabt, oh_p, preferred_element_type=jnp.float32)
    nidx = neg_ref[0, 0, :]
    oh_n = (nidx[None, :] == iota_k).astype(jnp.float32)
    nembt_ref[...] = jnp.dot(itabt, oh_n, preferred_element_type=jnp.float32)


def _tc_call(itabt, pos3, neg3, pugg):
    return pl.pallas_call(
        _tc_body,
        grid=(GRID,),
        in_specs=[
            pl.BlockSpec((D, NI1), lambda i: (0, 0)),
            pl.BlockSpec((1, 1, BB), lambda i: (i, 0, 0)),
            pl.BlockSpec((1, 1, BB), lambda i: (i, 0, 0)),
            pl.BlockSpec((BB, GW), lambda i: (i, 0)),
        ],
        out_specs=[
            pl.BlockSpec((D, BB), lambda i: (0, i)),
            pl.BlockSpec((D, BB), lambda i: (0, i)),
            pl.BlockSpec((D, BB), lambda i: (0, i)),
            pl.BlockSpec((D, BB), lambda i: (0, i)),
        ],
        out_shape=[
            jax.ShapeDtypeStruct((D, B), jnp.float32),
            jax.ShapeDtypeStruct((D, B), jnp.float32),
            jax.ShapeDtypeStruct((D, B), jnp.float32),
            jax.ShapeDtypeStruct((D, B), jnp.float32),
        ],
    )(itabt, pos3, neg3, pugg)


def _agg_body(utt_ref, bmu_ref, itabt_ref, bmi_ref, aggut_ref, aggit_ref):
    aggut_ref[...] = utt_ref[...] * bmu_ref[...]

    @pl.when(pl.program_id(0) == 0)
    def _():
        aggit_ref[...] = itabt_ref[...] * bmi_ref[:, :NI1]


def _agg_call(utt, bmu, itabt, bmi):
    return pl.pallas_call(
        _agg_body,
        grid=(AGRID,),
        in_specs=[
            pl.BlockSpec((D, AB), lambda i: (0, i)),
            pl.BlockSpec((1, AB), lambda i: (0, i)),
            pl.BlockSpec((D, NI1), lambda i: (0, 0)),
            pl.BlockSpec((1, NIW * I_RANGE), lambda i: (0, 0)),
        ],
        out_specs=[
            pl.BlockSpec((D, AB), lambda i: (0, i)),
            pl.BlockSpec((D, NI1), lambda i: (0, 0)),
        ],
        out_shape=[
            jax.ShapeDtypeStruct((D, NU), jnp.float32),
            jax.ShapeDtypeStruct((D, NI1), jnp.float32),
        ],
    )(utt, bmu, itabt, bmi)


@jax.jit
def kernel(user, pos, neg, train_label, user_table, item_table, agg_user, agg_item):
    del agg_user, agg_item
    labt = jnp.transpose(train_label)
    itabt = jnp.transpose(item_table)
    utt = jnp.transpose(user_table)
    (pug,) = _prep_call(labt, item_table, utt)
    pugg, bm_u, bm_i = _sc_call(user, pos, neg, pug)
    pos3 = pos.reshape(GRID, 1, BB)
    neg3 = neg.reshape(GRID, 1, BB)
    comt, pembt, nembt, uembt = _tc_call(itabt, pos3, neg3, pugg)
    aggut, aggit = _agg_call(utt, bm_u, itabt, bm_i)
    return (jnp.transpose(uembt), jnp.transpose(pembt), jnp.transpose(nembt),
            jnp.transpose(comt), jnp.transpose(aggut), jnp.transpose(aggit))


# speedup vs baseline: 10.3598x; 1.0300x over previous
import jax
import jax.numpy as jnp
from jax import lax
from jax.experimental import pallas as pl
from jax.experimental.pallas import tpu as pltpu
from jax.experimental.pallas import tpu_sc as plsc

B = 16384
D = 64
NU = 100000
NI1 = 1001
GW = 128

NW = 32
BPW = B // NW
U_RANGE = 3200
I_RANGE = 128
NIW = 8
LANES = 16

WCH = 128
NWCH = BPW // WCH

PB = 4096
PGRID = 25
NUP = PGRID * PB


def _prep_body(labt_ref, itab_ref, utt_ref, pug_ref):
    labt = labt_ref[...]
    pblk = lax.dot_general(labt, itab_ref[...], (((0,), (0,)), ((), ())),
                           preferred_element_type=jnp.float32)
    num = jnp.sum(labt, axis=0)[:, None]
    urow = jnp.transpose(utt_ref[...])
    pug_ref[...] = jnp.concatenate([pblk / num, urow], axis=1)


def _prep_call(labt, item_table, utt):
    return pl.pallas_call(
        _prep_body,
        grid=(PGRID,),
        in_specs=[
            pl.BlockSpec((NI1, PB), lambda i: (0, i)),
            pl.BlockSpec((NI1, D), lambda i: (0, 0)),
            pl.BlockSpec((D, PB), lambda i: (0, i)),
        ],
        out_specs=[pl.BlockSpec((PB, GW), lambda i: (i, 0))],
        out_shape=[jax.ShapeDtypeStruct((NUP, GW), jnp.float32)],
    )(labt, item_table, utt)


def _pipelined_gather(src_ref, dst_hbm, idx_ref, nchunks, chunk, base,
                      bufs, gsems, wsems):
    def g(c):
        return pltpu.make_async_copy(
            src_ref.at[idx_ref.at[c]], bufs[c % 2], gsems[c % 2])

    def w(c):
        return pltpu.make_async_copy(
            bufs[c % 2], dst_hbm.at[pl.ds(base + c * chunk, chunk)],
            wsems[c % 2])

    for c in range(nchunks):
        if c >= 2:
            w(c - 2).wait()
        g(c).start()
        if c >= 1:
            g(c - 1).wait()
            w(c - 1).start()
    g(nchunks - 1).wait()
    w(nchunks - 1).start()
    w(nchunks - 2).wait()
    w(nchunks - 1).wait()


def _sc_body(user_hbm, pos_hbm, neg_hbm, pug_hbm,
             pugg, bm_user, bm_item,
             ubuf, bmu, bmi, idxw, wbuf0, wbuf1,
             gsem0, gsem1, wsem0, wsem1):
    nc = 2
    wid = lax.axis_index("s") * nc + lax.axis_index("c")
    base = wid * BPW

    for j in range(NWCH):
        pltpu.sync_copy(user_hbm.at[pl.ds(base + j * WCH, WCH)], idxw.at[j])

    _pipelined_gather(pug_hbm, pugg, idxw, NWCH, WCH, base,
                      (wbuf0, wbuf1), (gsem0, gsem1), (wsem0, wsem1))

    lo_u = wid * U_RANGE
    zeros16 = jnp.zeros((LANES,), jnp.float32)
    ones16 = jnp.ones((LANES,), jnp.float32)

    pltpu.sync_copy(user_hbm, ubuf)

    def zero_bmu(i, _):
        bmu[pl.ds(i * LANES, LANES)] = zeros16
        return 0

    lax.fori_loop(0, U_RANGE // LANES, zero_bmu, 0)

    def mark_u(i, _):
        v = ubuf[pl.ds(i * LANES, LANES)]
        m = (v >= lo_u) & (v < lo_u + U_RANGE)
        plsc.store_scatter(bmu, [v - lo_u], ones16, mask=m)
        return 0

    lax.fori_loop(0, B // LANES, mark_u, 0)
    pltpu.sync_copy(bmu, bm_user.at[0, pl.ds(wid * U_RANGE, U_RANGE)])

    lo_i = wid * I_RANGE

    def zero_bmi(i, _):
        bmi[pl.ds(i * LANES, LANES)] = zeros16
        return 0

    def mark_i(i, _):
        v = ubuf[pl.ds(i * LANES, LANES)]
        m = (v >= lo_i) & (v < lo_i + I_RANGE)
        plsc.store_scatter(bmi, [v - lo_i], ones16, mask=m)
        return 0

    lax.fori_loop(0, I_RANGE // LANES, zero_bmi, 0)
    pltpu.sync_copy(pos_hbm, ubuf)

    @pl.when(wid < NIW)
    def _():
        lax.fori_loop(0, B // LANES, mark_i, 0)

    pltpu.sync_copy(neg_hbm, ubuf)

    @pl.when(wid < NIW)
    def _():
        lax.fori_loop(0, B // LANES, mark_i, 0)
        pltpu.sync_copy(bmi, bm_item.at[0, pl.ds(wid * I_RANGE, I_RANGE)])


def _sc_call(user, pos, neg, pug):
    mesh = plsc.VectorSubcoreMesh(core_axis_name="c", subcore_axis_name="s")
    f = pl.kernel(
        _sc_body,
        out_type=[
            jax.ShapeDtypeStruct((B, GW), jnp.float32),
            jax.ShapeDtypeStruct((1, NW * U_RANGE), jnp.float32),
            jax.ShapeDtypeStruct((1, NIW * I_RANGE), jnp.float32),
        ],
        mesh=mesh,
        scratch_types=[
            pltpu.VMEM((B,), jnp.int32),
            pltpu.VMEM((U_RANGE,), jnp.float32),
            pltpu.VMEM((I_RANGE,), jnp.float32),
            pltpu.VMEM((NWCH, WCH), jnp.int32),
            pltpu.VMEM((WCH, GW), jnp.float32),
            pltpu.VMEM((WCH, GW), jnp.float32),
            pltpu.SemaphoreType.DMA,
            pltpu.SemaphoreType.DMA,
            pltpu.SemaphoreType.DMA,
            pltpu.SemaphoreType.DMA,
        ],
        compiler_params=pltpu.CompilerParams(needs_layout_passes=False),
    )
    return f(user, pos, neg, pug)


BB = 256
GRID = B // BB
AB = 4096
AGRID = 25


def _tc_body(itabt_ref, pos_ref, neg_ref, pugg_ref,
             comt_ref, pembt_ref, nembt_ref, uembt_ref):
    itabt = itabt_ref[...]
    t = jnp.transpose(pugg_ref[...])
    comt_ref[...] = t[:D, :]
    uembt_ref[...] = t[D:GW, :]

    iota_k = lax.broadcasted_iota(jnp.int32, (NI1, BB), 0)
    pidx = pos_ref[0, 0, :]
    oh_p = (pidx[None, :] == iota_k).astype(jnp.float32)
    pembt_ref[...] = jnp.dot(itabt, oh_p, preferred_element_type=jnp.float32)
    nidx = neg_ref[0, 0, :]
    oh_n = (nidx[None, :] == iota_k).astype(jnp.float32)
    nembt_ref[...] = jnp.dot(itabt, oh_n, preferred_element_type=jnp.float32)


def _tc_call(itabt, pos3, neg3, pugg):
    return pl.pallas_call(
        _tc_body,
        grid=(GRID,),
        in_specs=[
            pl.BlockSpec((D, NI1), lambda i: (0, 0)),
            pl.BlockSpec((1, 1, BB), lambda i: (i, 0, 0)),
            pl.BlockSpec((1, 1, BB), lambda i: (i, 0, 0)),
            pl.BlockSpec((BB, GW), lambda i: (i, 0)),
        ],
        out_specs=[
            pl.BlockSpec((D, BB), lambda i: (0, i)),
            pl.BlockSpec((D, BB), lambda i: (0, i)),
            pl.BlockSpec((D, BB), lambda i: (0, i)),
            pl.BlockSpec((D, BB), lambda i: (0, i)),
        ],
        out_shape=[
            jax.ShapeDtypeStruct((D, B), jnp.float32),
            jax.ShapeDtypeStruct((D, B), jnp.float32),
            jax.ShapeDtypeStruct((D, B), jnp.float32),
            jax.ShapeDtypeStruct((D, B), jnp.float32),
        ],
    )(itabt, pos3, neg3, pugg)


def _agg_body(utt_ref, bmu_ref, itabt_ref, bmi_ref, aggut_ref, aggit_ref):
    aggut_ref[...] = utt_ref[...] * bmu_ref[...]

    @pl.when(pl.program_id(0) == 0)
    def _():
        aggit_ref[...] = itabt_ref[...] * bmi_ref[:, :NI1]


def _agg_call(utt, bmu, itabt, bmi):
    return pl.pallas_call(
        _agg_body,
        grid=(AGRID,),
        in_specs=[
            pl.BlockSpec((D, AB), lambda i: (0, i)),
            pl.BlockSpec((1, AB), lambda i: (0, i)),
            pl.BlockSpec((D, NI1), lambda i: (0, 0)),
            pl.BlockSpec((1, NIW * I_RANGE), lambda i: (0, 0)),
        ],
        out_specs=[
            pl.BlockSpec((D, AB), lambda i: (0, i)),
            pl.BlockSpec((D, NI1), lambda i: (0, 0)),
        ],
        out_shape=[
            jax.ShapeDtypeStruct((D, NU), jnp.float32),
            jax.ShapeDtypeStruct((D, NI1), jnp.float32),
        ],
    )(utt, bmu, itabt, bmi)


@jax.jit
def kernel(user, pos, neg, train_label, user_table, item_table, agg_user, agg_item):
    del agg_user, agg_item
    labt = jnp.transpose(train_label)
    itabt = jnp.transpose(item_table)
    utt = jnp.transpose(user_table)
    (pug,) = _prep_call(labt, item_table, utt)
    pugg, bm_u, bm_i = _sc_call(user, pos, neg, pug)
    pos3 = pos.reshape(GRID, 1, BB)
    neg3 = neg.reshape(GRID, 1, BB)
    comt, pembt, nembt, uembt = _tc_call(itabt, pos3, neg3, pugg)
    aggut, aggit = _agg_call(utt, bm_u, itabt, bm_i)
    return (jnp.transpose(uembt), jnp.transpose(pembt), jnp.transpose(nembt),
            jnp.transpose(comt), jnp.transpose(aggut), jnp.transpose(aggit))


# speedup vs baseline: 11.6434x; 1.1239x over previous
import jax
import jax.numpy as jnp
from jax import lax
from jax.experimental import pallas as pl
from jax.experimental.pallas import tpu as pltpu
from jax.experimental.pallas import tpu_sc as plsc

B = 16384
D = 64
NU = 100000
NI1 = 1001
GW = 128

NW = 32
BPW = B // NW
U_RANGE = 3200
I_RANGE = 128
NIW = 8
LANES = 16

WCH = 128
NWCH = BPW // WCH

PB = 4096
PGRID = 25
NUP = PGRID * PB


def _prep_body(labt_ref, itab_ref, utt_ref, pug_ref):
    labt = labt_ref[...]
    pblk = lax.dot_general(labt, itab_ref[...], (((0,), (0,)), ((), ())),
                           preferred_element_type=jnp.float32)
    num = jnp.sum(labt, axis=0)[:, None]
    urow = jnp.transpose(utt_ref[...])
    pug_ref[...] = jnp.concatenate([pblk / num, urow], axis=1)


def _prep_call(labt, item_table, utt):
    return pl.pallas_call(
        _prep_body,
        grid=(PGRID,),
        in_specs=[
            pl.BlockSpec((NI1, PB), lambda i: (0, i)),
            pl.BlockSpec((NI1, D), lambda i: (0, 0)),
            pl.BlockSpec((D, PB), lambda i: (0, i)),
        ],
        out_specs=[pl.BlockSpec((PB, GW), lambda i: (i, 0))],
        out_shape=[jax.ShapeDtypeStruct((NUP, GW), jnp.float32)],
    )(labt, item_table, utt)


def _pipelined_gather(src_ref, dst_hbm, idx_ref, nchunks, chunk, base,
                      bufs, gsems, wsems):
    def g(c):
        return pltpu.make_async_copy(
            src_ref.at[idx_ref.at[c]], bufs[c % 2], gsems[c % 2])

    def w(c):
        return pltpu.make_async_copy(
            bufs[c % 2], dst_hbm.at[pl.ds(base + c * chunk, chunk)],
            wsems[c % 2])

    for c in range(nchunks):
        if c >= 2:
            w(c - 2).wait()
        g(c).start()
        if c >= 1:
            g(c - 1).wait()
            w(c - 1).start()
    g(nchunks - 1).wait()
    w(nchunks - 1).start()
    w(nchunks - 2).wait()
    w(nchunks - 1).wait()


def _sc_gather_body(user_hbm, pug_hbm,
                    pugg,
                    idxw, wbuf0, wbuf1,
                    gsem0, gsem1, wsem0, wsem1):
    nc = 2
    wid = lax.axis_index("s") * nc + lax.axis_index("c")
    base = wid * BPW

    for j in range(NWCH):
        pltpu.sync_copy(user_hbm.at[pl.ds(base + j * WCH, WCH)], idxw.at[j])

    _pipelined_gather(pug_hbm, pugg, idxw, NWCH, WCH, base,
                      (wbuf0, wbuf1), (gsem0, gsem1), (wsem0, wsem1))


def _sc_bitmap_body(user_hbm, pos_hbm, neg_hbm,
                    bm_user, bm_item,
                    ubuf, bmu, bmi):
    nc = 2
    wid = lax.axis_index("s") * nc + lax.axis_index("c")

    lo_u = wid * U_RANGE
    zeros16 = jnp.zeros((LANES,), jnp.float32)
    ones16 = jnp.ones((LANES,), jnp.float32)

    pltpu.sync_copy(user_hbm, ubuf)

    def zero_bmu(i, _):
        bmu[pl.ds(i * LANES, LANES)] = zeros16
        return 0

    lax.fori_loop(0, U_RANGE // LANES, zero_bmu, 0)

    def mark_u(i, _):
        v = ubuf[pl.ds(i * LANES, LANES)]
        m = (v >= lo_u) & (v < lo_u + U_RANGE)
        plsc.store_scatter(bmu, [v - lo_u], ones16, mask=m)
        return 0

    lax.fori_loop(0, B // LANES, mark_u, 0)
    pltpu.sync_copy(bmu, bm_user.at[0, pl.ds(wid * U_RANGE, U_RANGE)])

    lo_i = wid * I_RANGE

    def zero_bmi(i, _):
        bmi[pl.ds(i * LANES, LANES)] = zeros16
        return 0

    def mark_i(i, _):
        v = ubuf[pl.ds(i * LANES, LANES)]
        m = (v >= lo_i) & (v < lo_i + I_RANGE)
        plsc.store_scatter(bmi, [v - lo_i], ones16, mask=m)
        return 0

    lax.fori_loop(0, I_RANGE // LANES, zero_bmi, 0)
    pltpu.sync_copy(pos_hbm, ubuf)

    @pl.when(wid < NIW)
    def _():
        lax.fori_loop(0, B // LANES, mark_i, 0)

    pltpu.sync_copy(neg_hbm, ubuf)

    @pl.when(wid < NIW)
    def _():
        lax.fori_loop(0, B // LANES, mark_i, 0)
        pltpu.sync_copy(bmi, bm_item.at[0, pl.ds(wid * I_RANGE, I_RANGE)])


def _sc_gather_call(user, pug):
    mesh = plsc.VectorSubcoreMesh(core_axis_name="c", subcore_axis_name="s")
    f = pl.kernel(
        _sc_gather_body,
        out_type=[
            jax.ShapeDtypeStruct((B, GW), jnp.float32),
        ],
        mesh=mesh,
        scratch_types=[
            pltpu.VMEM((NWCH, WCH), jnp.int32),
            pltpu.VMEM((WCH, GW), jnp.float32),
            pltpu.VMEM((WCH, GW), jnp.float32),
            pltpu.SemaphoreType.DMA,
            pltpu.SemaphoreType.DMA,
            pltpu.SemaphoreType.DMA,
            pltpu.SemaphoreType.DMA,
        ],
        compiler_params=pltpu.CompilerParams(needs_layout_passes=False),
    )
    return f(user, pug)


def _sc_bitmap_call(user, pos, neg):
    mesh = plsc.VectorSubcoreMesh(core_axis_name="c", subcore_axis_name="s")
    f = pl.kernel(
        _sc_bitmap_body,
        out_type=[
            jax.ShapeDtypeStruct((1, NW * U_RANGE), jnp.float32),
            jax.ShapeDtypeStruct((1, NIW * I_RANGE), jnp.float32),
        ],
        mesh=mesh,
        scratch_types=[
            pltpu.VMEM((B,), jnp.int32),
            pltpu.VMEM((U_RANGE,), jnp.float32),
            pltpu.VMEM((I_RANGE,), jnp.float32),
        ],
        compiler_params=pltpu.CompilerParams(needs_layout_passes=False),
    )
    return f(user, pos, neg)


BB = 256
GRID = B // BB
AB = 4096
AGRID = 25


def _tc_body(itabt_ref, pos_ref, neg_ref, pugg_ref,
             comt_ref, pembt_ref, nembt_ref, uembt_ref):
    itabt = itabt_ref[...]
    t = jnp.transpose(pugg_ref[...])
    comt_ref[...] = t[:D, :]
    uembt_ref[...] = t[D:GW, :]

    iota_k = lax.broadcasted_iota(jnp.int32, (NI1, BB), 0)
    pidx = pos_ref[0, 0, :]
    oh_p = (pidx[None, :] == iota_k).astype(jnp.float32)
    pembt_ref[...] = jnp.dot(itabt, oh_p, preferred_element_type=jnp.float32)
    nidx = neg_ref[0, 0, :]
    oh_n = (nidx[None, :] == iota_k).astype(jnp.float32)
    nembt_ref[...] = jnp.dot(itabt, oh_n, preferred_element_type=jnp.float32)


def _tc_call(itabt, pos3, neg3, pugg):
    return pl.pallas_call(
        _tc_body,
        grid=(GRID,),
        in_specs=[
            pl.BlockSpec((D, NI1), lambda i: (0, 0)),
            pl.BlockSpec((1, 1, BB), lambda i: (i, 0, 0)),
            pl.BlockSpec((1, 1, BB), lambda i: (i, 0, 0)),
            pl.BlockSpec((BB, GW), lambda i: (i, 0)),
        ],
        out_specs=[
            pl.BlockSpec((D, BB), lambda i: (0, i)),
            pl.BlockSpec((D, BB), lambda i: (0, i)),
            pl.BlockSpec((D, BB), lambda i: (0, i)),
            pl.BlockSpec((D, BB), lambda i: (0, i)),
        ],
        out_shape=[
            jax.ShapeDtypeStruct((D, B), jnp.float32),
            jax.ShapeDtypeStruct((D, B), jnp.float32),
            jax.ShapeDtypeStruct((D, B), jnp.float32),
            jax.ShapeDtypeStruct((D, B), jnp.float32),
        ],
    )(itabt, pos3, neg3, pugg)


def _agg_body(utt_ref, bmu_ref, itabt_ref, bmi_ref, aggut_ref, aggit_ref):
    aggut_ref[...] = utt_ref[...] * bmu_ref[...]

    @pl.when(pl.program_id(0) == 0)
    def _():
        aggit_ref[...] = itabt_ref[...] * bmi_ref[:, :NI1]


def _agg_call(utt, bmu, itabt, bmi):
    return pl.pallas_call(
        _agg_body,
        grid=(AGRID,),
        in_specs=[
            pl.BlockSpec((D, AB), lambda i: (0, i)),
            pl.BlockSpec((1, AB), lambda i: (0, i)),
            pl.BlockSpec((D, NI1), lambda i: (0, 0)),
            pl.BlockSpec((1, NIW * I_RANGE), lambda i: (0, 0)),
        ],
        out_specs=[
            pl.BlockSpec((D, AB), lambda i: (0, i)),
            pl.BlockSpec((D, NI1), lambda i: (0, 0)),
        ],
        out_shape=[
            jax.ShapeDtypeStruct((D, NU), jnp.float32),
            jax.ShapeDtypeStruct((D, NI1), jnp.float32),
        ],
    )(utt, bmu, itabt, bmi)


@jax.jit
def kernel(user, pos, neg, train_label, user_table, item_table, agg_user, agg_item):
    del agg_user, agg_item
    labt = jnp.transpose(train_label)
    itabt = jnp.transpose(item_table)
    utt = jnp.transpose(user_table)
    bm_u, bm_i = _sc_bitmap_call(user, pos, neg)
    (pug,) = _prep_call(labt, item_table, utt)
    (pugg,) = _sc_gather_call(user, pug)
    pos3 = pos.reshape(GRID, 1, BB)
    neg3 = neg.reshape(GRID, 1, BB)
    comt, pembt, nembt, uembt = _tc_call(itabt, pos3, neg3, pugg)
    aggut, aggit = _agg_call(utt, bm_u, itabt, bm_i)
    return (jnp.transpose(uembt), jnp.transpose(pembt), jnp.transpose(nembt),
            jnp.transpose(comt), jnp.transpose(aggut), jnp.transpose(aggit))


# speedup vs baseline: 12.7684x; 1.0966x over previous
import jax
import jax.numpy as jnp
from jax import lax
from jax.experimental import pallas as pl
from jax.experimental.pallas import tpu as pltpu
from jax.experimental.pallas import tpu_sc as plsc

B = 16384
D = 64
NU = 100000
NI1 = 1001
GW = 128

NW = 32
BPW = B // NW
U_RANGE = 3200
I_RANGE = 128
NIW = 8
LANES = 16

WCH = 128
NWCH = BPW // WCH

PB = 4096
PGRID = 25
NUP = PGRID * PB


def _prep_body(labt_ref, itab_ref, utt_ref, pug_ref):
    labt = labt_ref[...]
    pblk = lax.dot_general(labt, itab_ref[...], (((0,), (0,)), ((), ())),
                           preferred_element_type=jnp.float32)
    num = jnp.sum(labt, axis=0)[:, None]
    urow = jnp.transpose(utt_ref[...])
    pug_ref[...] = jnp.concatenate([pblk / num, urow], axis=1)


def _prep_call(labt, item_table, utt):
    return pl.pallas_call(
        _prep_body,
        grid=(PGRID,),
        in_specs=[
            pl.BlockSpec((NI1, PB), lambda i: (0, i)),
            pl.BlockSpec((NI1, D), lambda i: (0, 0)),
            pl.BlockSpec((D, PB), lambda i: (0, i)),
        ],
        out_specs=[pl.BlockSpec((PB, GW), lambda i: (i, 0))],
        out_shape=[jax.ShapeDtypeStruct((NUP, GW), jnp.float32)],
    )(labt, item_table, utt)


def _pipelined_gather(src_ref, dst_hbm, idx_ref, nchunks, chunk, base,
                      bufs, gsems, wsems):
    def g(c):
        return pltpu.make_async_copy(
            src_ref.at[idx_ref.at[c]], bufs[c % 2], gsems[c % 2])

    def w(c):
        return pltpu.make_async_copy(
            bufs[c % 2], dst_hbm.at[pl.ds(base + c * chunk, chunk)],
            wsems[c % 2])

    for c in range(nchunks):
        if c >= 2:
            w(c - 2).wait()
        g(c).start()
        if c >= 1:
            g(c - 1).wait()
            w(c - 1).start()
    g(nchunks - 1).wait()
    w(nchunks - 1).start()
    w(nchunks - 2).wait()
    w(nchunks - 1).wait()


def _sc_gather_body(user_hbm, pug_hbm,
                    pugg,
                    idxw, wbuf0, wbuf1,
                    gsem0, gsem1, wsem0, wsem1):
    nc = 2
    wid = lax.axis_index("s") * nc + lax.axis_index("c")
    base = wid * BPW

    for j in range(NWCH):
        pltpu.sync_copy(user_hbm.at[pl.ds(base + j * WCH, WCH)], idxw.at[j])

    _pipelined_gather(pug_hbm, pugg, idxw, NWCH, WCH, base,
                      (wbuf0, wbuf1), (gsem0, gsem1), (wsem0, wsem1))


def _sc_bitmap_body(user_hbm, pos_hbm, neg_hbm,
                    bm_user, bm_item,
                    ubuf, bmu, bmi):
    nc = 2
    wid = lax.axis_index("s") * nc + lax.axis_index("c")

    lo_u = wid * U_RANGE
    zeros16 = jnp.zeros((LANES,), jnp.float32)
    ones16 = jnp.ones((LANES,), jnp.float32)

    pltpu.sync_copy(user_hbm, ubuf)

    def zero_bmu(i, _):
        bmu[pl.ds(i * LANES, LANES)] = zeros16
        return 0

    lax.fori_loop(0, U_RANGE // LANES, zero_bmu, 0)

    def mark_u(i, _):
        v = ubuf[pl.ds(i * LANES, LANES)]
        m = (v >= lo_u) & (v < lo_u + U_RANGE)
        plsc.store_scatter(bmu, [v - lo_u], ones16, mask=m)
        return 0

    lax.fori_loop(0, B // LANES, mark_u, 0)
    pltpu.sync_copy(bmu, bm_user.at[0, pl.ds(wid * U_RANGE, U_RANGE)])

    lo_i = wid * I_RANGE

    def zero_bmi(i, _):
        bmi[pl.ds(i * LANES, LANES)] = zeros16
        return 0

    def mark_i(i, _):
        v = ubuf[pl.ds(i * LANES, LANES)]
        m = (v >= lo_i) & (v < lo_i + I_RANGE)
        plsc.store_scatter(bmi, [v - lo_i], ones16, mask=m)
        return 0

    lax.fori_loop(0, I_RANGE // LANES, zero_bmi, 0)
    pltpu.sync_copy(pos_hbm, ubuf)

    @pl.when(wid < NIW)
    def _():
        lax.fori_loop(0, B // LANES, mark_i, 0)

    pltpu.sync_copy(neg_hbm, ubuf)

    @pl.when(wid < NIW)
    def _():
        lax.fori_loop(0, B // LANES, mark_i, 0)
        pltpu.sync_copy(bmi, bm_item.at[0, pl.ds(wid * I_RANGE, I_RANGE)])


def _sc_gather_call(user, pug):
    mesh = plsc.VectorSubcoreMesh(core_axis_name="c", subcore_axis_name="s")
    f = pl.kernel(
        _sc_gather_body,
        out_type=[
            jax.ShapeDtypeStruct((B, GW), jnp.float32),
        ],
        mesh=mesh,
        scratch_types=[
            pltpu.VMEM((NWCH, WCH), jnp.int32),
            pltpu.VMEM((WCH, GW), jnp.float32),
            pltpu.VMEM((WCH, GW), jnp.float32),
            pltpu.SemaphoreType.DMA,
            pltpu.SemaphoreType.DMA,
            pltpu.SemaphoreType.DMA,
            pltpu.SemaphoreType.DMA,
        ],
        compiler_params=pltpu.CompilerParams(needs_layout_passes=False),
    )
    return f(user, pug)


def _sc_bitmap_call(user, pos, neg):
    mesh = plsc.VectorSubcoreMesh(core_axis_name="c", subcore_axis_name="s")
    f = pl.kernel(
        _sc_bitmap_body,
        out_type=[
            jax.ShapeDtypeStruct((1, NW * U_RANGE), jnp.float32),
            jax.ShapeDtypeStruct((1, NIW * I_RANGE), jnp.float32),
        ],
        mesh=mesh,
        scratch_types=[
            pltpu.VMEM((B,), jnp.int32),
            pltpu.VMEM((U_RANGE,), jnp.float32),
            pltpu.VMEM((I_RANGE,), jnp.float32),
        ],
        compiler_params=pltpu.CompilerParams(needs_layout_passes=False),
    )
    return f(user, pos, neg)


BB = 512
GRID = B // BB
AB = 8192
AGRID = 13


def _tc_body(itabt_ref, pos_ref, neg_ref, pugg_ref,
             comt_ref, pembt_ref, nembt_ref, uembt_ref):
    itabt = itabt_ref[...]
    t = jnp.transpose(pugg_ref[...])
    comt_ref[...] = t[:D, :]
    uembt_ref[...] = t[D:GW, :]

    iota_k = lax.broadcasted_iota(jnp.int32, (NI1, BB), 0)
    pidx = pos_ref[0, 0, :]
    oh_p = (pidx[None, :] == iota_k).astype(jnp.float32)
    pembt_ref[...] = jnp.dot(itabt, oh_p, preferred_element_type=jnp.float32)
    nidx = neg_ref[0, 0, :]
    oh_n = (nidx[None, :] == iota_k).astype(jnp.float32)
    nembt_ref[...] = jnp.dot(itabt, oh_n, preferred_element_type=jnp.float32)


def _tc_call(itabt, pos3, neg3, pugg):
    return pl.pallas_call(
        _tc_body,
        grid=(GRID,),
        in_specs=[
            pl.BlockSpec((D, NI1), lambda i: (0, 0)),
            pl.BlockSpec((1, 1, BB), lambda i: (i, 0, 0)),
            pl.BlockSpec((1, 1, BB), lambda i: (i, 0, 0)),
            pl.BlockSpec((BB, GW), lambda i: (i, 0)),
        ],
        out_specs=[
            pl.BlockSpec((D, BB), lambda i: (0, i)),
            pl.BlockSpec((D, BB), lambda i: (0, i)),
            pl.BlockSpec((D, BB), lambda i: (0, i)),
            pl.BlockSpec((D, BB), lambda i: (0, i)),
        ],
        out_shape=[
            jax.ShapeDtypeStruct((D, B), jnp.float32),
            jax.ShapeDtypeStruct((D, B), jnp.float32),
            jax.ShapeDtypeStruct((D, B), jnp.float32),
            jax.ShapeDtypeStruct((D, B), jnp.float32),
        ],
    )(itabt, pos3, neg3, pugg)


def _agg_body(utt_ref, bmu_ref, itabt_ref, bmi_ref, aggut_ref, aggit_ref):
    aggut_ref[...] = utt_ref[...] * bmu_ref[...]

    @pl.when(pl.program_id(0) == 0)
    def _():
        aggit_ref[...] = itabt_ref[...] * bmi_ref[:, :NI1]


def _agg_call(utt, bmu, itabt, bmi):
    return pl.pallas_call(
        _agg_body,
        grid=(AGRID,),
        in_specs=[
            pl.BlockSpec((D, AB), lambda i: (0, i)),
            pl.BlockSpec((1, AB), lambda i: (0, i)),
            pl.BlockSpec((D, NI1), lambda i: (0, 0)),
            pl.BlockSpec((1, NIW * I_RANGE), lambda i: (0, 0)),
        ],
        out_specs=[
            pl.BlockSpec((D, AB), lambda i: (0, i)),
            pl.BlockSpec((D, NI1), lambda i: (0, 0)),
        ],
        out_shape=[
            jax.ShapeDtypeStruct((D, NU), jnp.float32),
            jax.ShapeDtypeStruct((D, NI1), jnp.float32),
        ],
    )(utt, bmu, itabt, bmi)


@jax.jit
def kernel(user, pos, neg, train_label, user_table, item_table, agg_user, agg_item):
    del agg_user, agg_item
    labt = jnp.transpose(train_label)
    itabt = jnp.transpose(item_table)
    utt = jnp.transpose(user_table)
    bm_u, bm_i = _sc_bitmap_call(user, pos, neg)
    (pug,) = _prep_call(labt, item_table, utt)
    (pugg,) = _sc_gather_call(user, pug)
    pos3 = pos.reshape(GRID, 1, BB)
    neg3 = neg.reshape(GRID, 1, BB)
    comt, pembt, nembt, uembt = _tc_call(itabt, pos3, neg3, pugg)
    aggut, aggit = _agg_call(utt, bm_u, itabt, bm_i)
    return (jnp.transpose(uembt), jnp.transpose(pembt), jnp.transpose(nembt),
            jnp.transpose(comt), jnp.transpose(aggut), jnp.transpose(aggit))


# speedup vs baseline: 13.3253x; 1.0436x over previous
import jax
import jax.numpy as jnp
from jax import lax
from jax.experimental import pallas as pl
from jax.experimental.pallas import tpu as pltpu
from jax.experimental.pallas import tpu_sc as plsc

B = 16384
D = 64
NU = 100000
NI1 = 1001
GW = 128

NW = 32
BPW = B // NW
U_RANGE = 3200
I_RANGE = 128
NIW = 8
LANES = 16

WCH = 128
NWCH = BPW // WCH

PB = 4096
PGRID = 25
NUP = PGRID * PB


def _prep_body(labt_ref, itab_ref, utt_ref, pug_ref):
    labt = labt_ref[...]
    pblk = lax.dot_general(labt, itab_ref[...], (((0,), (0,)), ((), ())),
                           preferred_element_type=jnp.float32)
    num = jnp.sum(labt, axis=0)[:, None]
    urow = jnp.transpose(utt_ref[...])
    pug_ref[...] = jnp.concatenate([pblk / num, urow], axis=1)


def _prep_call(labt, item_table, utt):
    return pl.pallas_call(
        _prep_body,
        grid=(PGRID,),
        in_specs=[
            pl.BlockSpec((NI1, PB), lambda i: (0, i)),
            pl.BlockSpec((NI1, D), lambda i: (0, 0)),
            pl.BlockSpec((D, PB), lambda i: (0, i)),
        ],
        out_specs=[pl.BlockSpec((PB, GW), lambda i: (i, 0))],
        out_shape=[jax.ShapeDtypeStruct((NUP, GW), jnp.float32)],
    )(labt, item_table, utt)


def _pipelined_gather(src_ref, dst_hbm, idx_ref, nchunks, chunk, base,
                      bufs, gsems, wsems):
    def g(c):
        return pltpu.make_async_copy(
            src_ref.at[idx_ref.at[c]], bufs[c % 2], gsems[c % 2])

    def w(c):
        return pltpu.make_async_copy(
            bufs[c % 2], dst_hbm.at[pl.ds(base + c * chunk, chunk)],
            wsems[c % 2])

    for c in range(nchunks):
        if c >= 2:
            w(c - 2).wait()
        g(c).start()
        if c >= 1:
            g(c - 1).wait()
            w(c - 1).start()
    g(nchunks - 1).wait()
    w(nchunks - 1).start()
    w(nchunks - 2).wait()
    w(nchunks - 1).wait()


def _sc_gather_body(user_hbm, pug_hbm,
                    pugg,
                    idxw, wbuf0, wbuf1,
                    gsem0, gsem1, wsem0, wsem1):
    nc = 2
    wid = lax.axis_index("s") * nc + lax.axis_index("c")
    base = wid * BPW

    for j in range(NWCH):
        pltpu.sync_copy(user_hbm.at[pl.ds(base + j * WCH, WCH)], idxw.at[j])

    _pipelined_gather(pug_hbm, pugg, idxw, NWCH, WCH, base,
                      (wbuf0, wbuf1), (gsem0, gsem1), (wsem0, wsem1))


def _sc_bitmap_body(user_hbm, pos_hbm, neg_hbm,
                    bm_user, bm_item,
                    ubuf, bmu, bmi):
    nc = 2
    wid = lax.axis_index("s") * nc + lax.axis_index("c")

    lo_u = wid * U_RANGE
    zeros16 = jnp.zeros((LANES,), jnp.float32)
    ones16 = jnp.ones((LANES,), jnp.float32)

    pltpu.sync_copy(user_hbm, ubuf)

    def zero_bmu(i, _):
        bmu[pl.ds(i * LANES, LANES)] = zeros16
        return 0

    lax.fori_loop(0, U_RANGE // LANES, zero_bmu, 0)

    def mark_u(i, _):
        v = ubuf[pl.ds(i * LANES, LANES)]
        m = (v >= lo_u) & (v < lo_u + U_RANGE)
        plsc.store_scatter(bmu, [v - lo_u], ones16, mask=m)
        return 0

    lax.fori_loop(0, B // LANES, mark_u, 0)
    pltpu.sync_copy(bmu, bm_user.at[0, pl.ds(wid * U_RANGE, U_RANGE)])

    lo_i = wid * I_RANGE

    def zero_bmi(i, _):
        bmi[pl.ds(i * LANES, LANES)] = zeros16
        return 0

    def mark_i(i, _):
        v = ubuf[pl.ds(i * LANES, LANES)]
        m = (v >= lo_i) & (v < lo_i + I_RANGE)
        plsc.store_scatter(bmi, [v - lo_i], ones16, mask=m)
        return 0

    lax.fori_loop(0, I_RANGE // LANES, zero_bmi, 0)
    pltpu.sync_copy(pos_hbm, ubuf)

    @pl.when(wid < NIW)
    def _():
        lax.fori_loop(0, B // LANES, mark_i, 0)

    pltpu.sync_copy(neg_hbm, ubuf)

    @pl.when(wid < NIW)
    def _():
        lax.fori_loop(0, B // LANES, mark_i, 0)
        pltpu.sync_copy(bmi, bm_item.at[0, pl.ds(wid * I_RANGE, I_RANGE)])


def _sc_gather_call(user, pug):
    mesh = plsc.VectorSubcoreMesh(core_axis_name="c", subcore_axis_name="s")
    f = pl.kernel(
        _sc_gather_body,
        out_type=[
            jax.ShapeDtypeStruct((B, GW), jnp.float32),
        ],
        mesh=mesh,
        scratch_types=[
            pltpu.VMEM((NWCH, WCH), jnp.int32),
            pltpu.VMEM((WCH, GW), jnp.float32),
            pltpu.VMEM((WCH, GW), jnp.float32),
            pltpu.SemaphoreType.DMA,
            pltpu.SemaphoreType.DMA,
            pltpu.SemaphoreType.DMA,
            pltpu.SemaphoreType.DMA,
        ],
        compiler_params=pltpu.CompilerParams(needs_layout_passes=False),
    )
    return f(user, pug)


def _sc_bitmap_call(user, pos, neg):
    mesh = plsc.VectorSubcoreMesh(core_axis_name="c", subcore_axis_name="s")
    f = pl.kernel(
        _sc_bitmap_body,
        out_type=[
            jax.ShapeDtypeStruct((1, NW * U_RANGE), jnp.float32),
            jax.ShapeDtypeStruct((1, NIW * I_RANGE), jnp.float32),
        ],
        mesh=mesh,
        scratch_types=[
            pltpu.VMEM((B,), jnp.int32),
            pltpu.VMEM((U_RANGE,), jnp.float32),
            pltpu.VMEM((I_RANGE,), jnp.float32),
        ],
        compiler_params=pltpu.CompilerParams(needs_layout_passes=False),
    )
    return f(user, pos, neg)


BB = 1024
GRID = B // BB
AB = 16384
AGRID = 7


def _tc_body(itabt_ref, pos_ref, neg_ref, pugg_ref,
             comt_ref, pembt_ref, nembt_ref, uembt_ref):
    itabt = itabt_ref[...]
    t = jnp.transpose(pugg_ref[...])
    comt_ref[...] = t[:D, :]
    uembt_ref[...] = t[D:GW, :]

    iota_k = lax.broadcasted_iota(jnp.int32, (NI1, BB), 0)
    pidx = pos_ref[0, 0, :]
    oh_p = (pidx[None, :] == iota_k).astype(jnp.float32)
    pembt_ref[...] = jnp.dot(itabt, oh_p, preferred_element_type=jnp.float32)
    nidx = neg_ref[0, 0, :]
    oh_n = (nidx[None, :] == iota_k).astype(jnp.float32)
    nembt_ref[...] = jnp.dot(itabt, oh_n, preferred_element_type=jnp.float32)


def _tc_call(itabt, pos3, neg3, pugg):
    return pl.pallas_call(
        _tc_body,
        grid=(GRID,),
        in_specs=[
            pl.BlockSpec((D, NI1), lambda i: (0, 0)),
            pl.BlockSpec((1, 1, BB), lambda i: (i, 0, 0)),
            pl.BlockSpec((1, 1, BB), lambda i: (i, 0, 0)),
            pl.BlockSpec((BB, GW), lambda i: (i, 0)),
        ],
        out_specs=[
            pl.BlockSpec((D, BB), lambda i: (0, i)),
            pl.BlockSpec((D, BB), lambda i: (0, i)),
            pl.BlockSpec((D, BB), lambda i: (0, i)),
            pl.BlockSpec((D, BB), lambda i: (0, i)),
        ],
        out_shape=[
            jax.ShapeDtypeStruct((D, B), jnp.float32),
            jax.ShapeDtypeStruct((D, B), jnp.float32),
            jax.ShapeDtypeStruct((D, B), jnp.float32),
            jax.ShapeDtypeStruct((D, B), jnp.float32),
        ],
    )(itabt, pos3, neg3, pugg)


def _agg_body(utt_ref, bmu_ref, itabt_ref, bmi_ref, aggut_ref, aggit_ref):
    aggut_ref[...] = utt_ref[...] * bmu_ref[...]

    @pl.when(pl.program_id(0) == 0)
    def _():
        aggit_ref[...] = itabt_ref[...] * bmi_ref[:, :NI1]


def _agg_call(utt, bmu, itabt, bmi):
    return pl.pallas_call(
        _agg_body,
        grid=(AGRID,),
        in_specs=[
            pl.BlockSpec((D, AB), lambda i: (0, i)),
            pl.BlockSpec((1, AB), lambda i: (0, i)),
            pl.BlockSpec((D, NI1), lambda i: (0, 0)),
            pl.BlockSpec((1, NIW * I_RANGE), lambda i: (0, 0)),
        ],
        out_specs=[
            pl.BlockSpec((D, AB), lambda i: (0, i)),
            pl.BlockSpec((D, NI1), lambda i: (0, 0)),
        ],
        out_shape=[
            jax.ShapeDtypeStruct((D, NU), jnp.float32),
            jax.ShapeDtypeStruct((D, NI1), jnp.float32),
        ],
    )(utt, bmu, itabt, bmi)


@jax.jit
def kernel(user, pos, neg, train_label, user_table, item_table, agg_user, agg_item):
    del agg_user, agg_item
    labt = jnp.transpose(train_label)
    itabt = jnp.transpose(item_table)
    utt = jnp.transpose(user_table)
    bm_u, bm_i = _sc_bitmap_call(user, pos, neg)
    (pug,) = _prep_call(labt, item_table, utt)
    (pugg,) = _sc_gather_call(user, pug)
    pos3 = pos.reshape(GRID, 1, BB)
    neg3 = neg.reshape(GRID, 1, BB)
    comt, pembt, nembt, uembt = _tc_call(itabt, pos3, neg3, pugg)
    aggut, aggit = _agg_call(utt, bm_u, itabt, bm_i)
    return (jnp.transpose(uembt), jnp.transpose(pembt), jnp.transpose(nembt),
            jnp.transpose(comt), jnp.transpose(aggut), jnp.transpose(aggit))


# speedup vs baseline: 13.4628x; 1.0103x over previous
import jax
import jax.numpy as jnp
from jax import lax
from jax.experimental import pallas as pl
from jax.experimental.pallas import tpu as pltpu
from jax.experimental.pallas import tpu_sc as plsc

B = 16384
D = 64
NU = 100000
NI1 = 1001
GW = 128

NW = 32
BPW = B // NW
U_RANGE = 3200
I_RANGE = 128
NIW = 8
LANES = 16

WCH = 128
NWCH = BPW // WCH

PB = 4096
PGRID = 25
NUP = PGRID * PB


def _prep_body(labt_ref, itab_ref, utt_ref, pug_ref):
    labt = labt_ref[...]
    pblk = lax.dot_general(labt, itab_ref[...], (((0,), (0,)), ((), ())),
                           preferred_element_type=jnp.float32)
    num = jnp.sum(labt, axis=0)[:, None]
    urow = jnp.transpose(utt_ref[...])
    pug_ref[...] = jnp.concatenate([pblk / num, urow], axis=1)


def _prep_call(labt, item_table, utt):
    return pl.pallas_call(
        _prep_body,
        grid=(PGRID,),
        in_specs=[
            pl.BlockSpec((NI1, PB), lambda i: (0, i)),
            pl.BlockSpec((NI1, D), lambda i: (0, 0)),
            pl.BlockSpec((D, PB), lambda i: (0, i)),
        ],
        out_specs=[pl.BlockSpec((PB, GW), lambda i: (i, 0))],
        out_shape=[jax.ShapeDtypeStruct((NUP, GW), jnp.float32)],
    )(labt, item_table, utt)


def _pipelined_gather(src_ref, dst_hbm, idx_ref, nchunks, chunk, base,
                      bufs, gsems, wsems):
    def g(c):
        return pltpu.make_async_copy(
            src_ref.at[idx_ref.at[c]], bufs[c % 2], gsems[c % 2])

    def w(c):
        return pltpu.make_async_copy(
            bufs[c % 2], dst_hbm.at[pl.ds(base + c * chunk, chunk)],
            wsems[c % 2])

    for c in range(nchunks):
        if c >= 2:
            w(c - 2).wait()
        g(c).start()
        if c >= 1:
            g(c - 1).wait()
            w(c - 1).start()
    g(nchunks - 1).wait()
    w(nchunks - 1).start()
    w(nchunks - 2).wait()
    w(nchunks - 1).wait()


def _sc_gather_body(user_hbm, pug_hbm,
                    pugg,
                    idxw, wbuf0, wbuf1,
                    gsem0, gsem1, wsem0, wsem1):
    nc = 2
    wid = lax.axis_index("s") * nc + lax.axis_index("c")
    base = wid * BPW

    for j in range(NWCH):
        pltpu.sync_copy(user_hbm.at[pl.ds(base + j * WCH, WCH)], idxw.at[j])

    _pipelined_gather(pug_hbm, pugg, idxw, NWCH, WCH, base,
                      (wbuf0, wbuf1), (gsem0, gsem1), (wsem0, wsem1))


def _sc_bitmap_body(user_hbm, pos_hbm, neg_hbm,
                    bm_user, bm_item,
                    ubuf, bmu, bmi):
    nc = 2
    wid = lax.axis_index("s") * nc + lax.axis_index("c")

    lo_u = wid * U_RANGE
    zeros16 = jnp.zeros((LANES,), jnp.float32)
    ones16 = jnp.ones((LANES,), jnp.float32)

    pltpu.sync_copy(user_hbm, ubuf)

    def zero_bmu(i, _):
        bmu[pl.ds(i * LANES, LANES)] = zeros16
        return 0

    lax.fori_loop(0, U_RANGE // LANES, zero_bmu, 0)

    def mark_u(i, _):
        v = ubuf[pl.ds(i * LANES, LANES)]
        m = (v >= lo_u) & (v < lo_u + U_RANGE)
        plsc.store_scatter(bmu, [v - lo_u], ones16, mask=m)
        return 0

    lax.fori_loop(0, B // LANES, mark_u, 0)
    pltpu.sync_copy(bmu, bm_user.at[0, pl.ds(wid * U_RANGE, U_RANGE)])

    lo_i = wid * I_RANGE

    def zero_bmi(i, _):
        bmi[pl.ds(i * LANES, LANES)] = zeros16
        return 0

    def mark_i(i, _):
        v = ubuf[pl.ds(i * LANES, LANES)]
        m = (v >= lo_i) & (v < lo_i + I_RANGE)
        plsc.store_scatter(bmi, [v - lo_i], ones16, mask=m)
        return 0

    lax.fori_loop(0, I_RANGE // LANES, zero_bmi, 0)
    pltpu.sync_copy(pos_hbm, ubuf)

    @pl.when(wid < NIW)
    def _():
        lax.fori_loop(0, B // LANES, mark_i, 0)

    pltpu.sync_copy(neg_hbm, ubuf)

    @pl.when(wid < NIW)
    def _():
        lax.fori_loop(0, B // LANES, mark_i, 0)
        pltpu.sync_copy(bmi, bm_item.at[0, pl.ds(wid * I_RANGE, I_RANGE)])


def _sc_gather_call(user, pug):
    mesh = plsc.VectorSubcoreMesh(core_axis_name="c", subcore_axis_name="s")
    f = pl.kernel(
        _sc_gather_body,
        out_type=[
            jax.ShapeDtypeStruct((B, GW), jnp.float32),
        ],
        mesh=mesh,
        scratch_types=[
            pltpu.VMEM((NWCH, WCH), jnp.int32),
            pltpu.VMEM((WCH, GW), jnp.float32),
            pltpu.VMEM((WCH, GW), jnp.float32),
            pltpu.SemaphoreType.DMA,
            pltpu.SemaphoreType.DMA,
            pltpu.SemaphoreType.DMA,
            pltpu.SemaphoreType.DMA,
        ],
        compiler_params=pltpu.CompilerParams(needs_layout_passes=False),
    )
    return f(user, pug)


def _sc_bitmap_call(user, pos, neg):
    mesh = plsc.VectorSubcoreMesh(core_axis_name="c", subcore_axis_name="s")
    f = pl.kernel(
        _sc_bitmap_body,
        out_type=[
            jax.ShapeDtypeStruct((1, NW * U_RANGE), jnp.float32),
            jax.ShapeDtypeStruct((1, NIW * I_RANGE), jnp.float32),
        ],
        mesh=mesh,
        scratch_types=[
            pltpu.VMEM((B,), jnp.int32),
            pltpu.VMEM((U_RANGE,), jnp.float32),
            pltpu.VMEM((I_RANGE,), jnp.float32),
        ],
        compiler_params=pltpu.CompilerParams(needs_layout_passes=False),
    )
    return f(user, pos, neg)


BB = 2048
GRID = B // BB
AB = 25600
AGRID = 4


def _tc_body(itabt_ref, pos_ref, neg_ref, pugg_ref,
             comt_ref, pembt_ref, nembt_ref, uembt_ref):
    itabt = itabt_ref[...]
    t = jnp.transpose(pugg_ref[...])
    comt_ref[...] = t[:D, :]
    uembt_ref[...] = t[D:GW, :]

    iota_k = lax.broadcasted_iota(jnp.int32, (NI1, BB), 0)
    pidx = pos_ref[0, 0, :]
    oh_p = (pidx[None, :] == iota_k).astype(jnp.float32)
    pembt_ref[...] = jnp.dot(itabt, oh_p, preferred_element_type=jnp.float32)
    nidx = neg_ref[0, 0, :]
    oh_n = (nidx[None, :] == iota_k).astype(jnp.float32)
    nembt_ref[...] = jnp.dot(itabt, oh_n, preferred_element_type=jnp.float32)


def _tc_call(itabt, pos3, neg3, pugg):
    return pl.pallas_call(
        _tc_body,
        grid=(GRID,),
        in_specs=[
            pl.BlockSpec((D, NI1), lambda i: (0, 0)),
            pl.BlockSpec((1, 1, BB), lambda i: (i, 0, 0)),
            pl.BlockSpec((1, 1, BB), lambda i: (i, 0, 0)),
            pl.BlockSpec((BB, GW), lambda i: (i, 0)),
        ],
        out_specs=[
            pl.BlockSpec((D, BB), lambda i: (0, i)),
            pl.BlockSpec((D, BB), lambda i: (0, i)),
            pl.BlockSpec((D, BB), lambda i: (0, i)),
            pl.BlockSpec((D, BB), lambda i: (0, i)),
        ],
        out_shape=[
            jax.ShapeDtypeStruct((D, B), jnp.float32),
            jax.ShapeDtypeStruct((D, B), jnp.float32),
            jax.ShapeDtypeStruct((D, B), jnp.float32),
            jax.ShapeDtypeStruct((D, B), jnp.float32),
        ],
    )(itabt, pos3, neg3, pugg)


def _agg_body(utt_ref, bmu_ref, itabt_ref, bmi_ref, aggut_ref, aggit_ref):
    aggut_ref[...] = utt_ref[...] * bmu_ref[...]

    @pl.when(pl.program_id(0) == 0)
    def _():
        aggit_ref[...] = itabt_ref[...] * bmi_ref[:, :NI1]


def _agg_call(utt, bmu, itabt, bmi):
    return pl.pallas_call(
        _agg_body,
        grid=(AGRID,),
        in_specs=[
            pl.BlockSpec((D, AB), lambda i: (0, i)),
            pl.BlockSpec((1, AB), lambda i: (0, i)),
            pl.BlockSpec((D, NI1), lambda i: (0, 0)),
            pl.BlockSpec((1, NIW * I_RANGE), lambda i: (0, 0)),
        ],
        out_specs=[
            pl.BlockSpec((D, AB), lambda i: (0, i)),
            pl.BlockSpec((D, NI1), lambda i: (0, 0)),
        ],
        out_shape=[
            jax.ShapeDtypeStruct((D, NU), jnp.float32),
            jax.ShapeDtypeStruct((D, NI1), jnp.float32),
        ],
    )(utt, bmu, itabt, bmi)


@jax.jit
def kernel(user, pos, neg, train_label, user_table, item_table, agg_user, agg_item):
    del agg_user, agg_item
    labt = jnp.transpose(train_label)
    itabt = jnp.transpose(item_table)
    utt = jnp.transpose(user_table)
    bm_u, bm_i = _sc_bitmap_call(user, pos, neg)
    (pug,) = _prep_call(labt, item_table, utt)
    (pugg,) = _sc_gather_call(user, pug)
    pos3 = pos.reshape(GRID, 1, BB)
    neg3 = neg.reshape(GRID, 1, BB)
    comt, pembt, nembt, uembt = _tc_call(itabt, pos3, neg3, pugg)
    aggut, aggit = _agg_call(utt, bm_u, itabt, bm_i)
    return (jnp.transpose(uembt), jnp.transpose(pembt), jnp.transpose(nembt),
            jnp.transpose(comt), jnp.transpose(aggut), jnp.transpose(aggit))
